# Initial kernel scaffold; baseline (speedup 1.0000x reference)
import functools

import numpy as np
import jax
import jax.numpy as jnp
from jax import lax
from jax.experimental import pallas as pl
from jax.experimental.pallas import tpu as pltpu
from jax.experimental.pallas import tpu_sc as plsc

F32 = jnp.float32

NC, NS = 2, 16
NW = NC * NS
V_, E_ = 10000, 320000
VP = 10240
EPAD = 327680
CH = 128
EPW = EPAD // NW
NCH = EPW // CH
ZR = VP // NS
BLK_E = 2560
GRID_E = E_ // BLK_E
BLK_V = 2048
GRID_V = VP // BLK_V
BLK_VO = 2000
GRID_VO = V_ // BLK_VO

_mesh = plsc.VectorSubcoreMesh(core_axis_name="c", subcore_axis_name="s")


def _h0_body(x_ref, w_ref, b_ref, o_ref):
    o_ref[...] = (
        jnp.dot(x_ref[...], w_ref[...], preferred_element_type=F32) + b_ref[...]
    )


def _edge_matvec(mT, gz):
    B = gz.shape[0]
    z = jnp.transpose(gz, (1, 0))[:8, :]
    zrep = jnp.broadcast_to(z.reshape(8, 1, B), (8, 8, B)).reshape(64, B)
    u = (mT * zrep).reshape(8, 8, B).sum(axis=0)
    urep = jnp.broadcast_to(u.reshape(1, 8, B), (8, 8, B)).reshape(64, B)
    w = (mT * urep).reshape(8, 8, B).sum(axis=1)
    zpad = jnp.zeros((B, 8), F32)
    U = jnp.concatenate([jnp.transpose(-u, (1, 0)), zpad], axis=1)
    W = jnp.concatenate([jnp.transpose(w, (1, 0)), zpad], axis=1)
    return U, W


def _maps_mv1_body(gs_ref, gt_ref, w1at_ref, w1bt_ref, w2pt_ref,
                   mt_ref, u_ref, w_ref):
    gs = gs_ref[...]
    gt = gt_ref[...]
    hidT = lax.dot_general(w1at_ref[...], gs, (((1,), (1,)), ((), ())),
                           preferred_element_type=F32)
    hidT = hidT + lax.dot_general(w1bt_ref[...], gt, (((1,), (1,)), ((), ())),
                                  preferred_element_type=F32)
    hidT = jnp.tanh(hidT)
    mT = jnp.tanh(lax.dot_general(w2pt_ref[...], hidT, (((1,), (0,)), ((), ())),
                                  preferred_element_type=F32))
    mt_ref[...] = mT
    U, W = _edge_matvec(mT, gt)
    u_ref[...] = U
    w_ref[...] = W


def _mv2_body(mt_ref, gz_ref, u_ref, w_ref):
    U, W = _edge_matvec(mt_ref[...], gz_ref[...])
    u_ref[...] = U
    w_ref[...] = W


def _cmb_body(acc_ref, h_ref, wl_ref, bl_ref, o_ref):
    a = acc_ref[0] + acc_ref[1]
    o_ref[...] = jnp.maximum(
        a + jnp.dot(h_ref[...], wl_ref[...], preferred_element_type=F32)
        + bl_ref[...], 0.0)


def _cmb_out_body(acc_ref, h_ref, wl_ref, bl_ref, wo_ref, bo_ref, o_ref):
    a = acc_ref[0] + acc_ref[1]
    h2 = jnp.maximum(
        a + jnp.dot(h_ref[...], wl_ref[...], preferred_element_type=F32)
        + bl_ref[...], 0.0)
    o_ref[...] = jnp.dot(h2, wo_ref[...], preferred_element_type=F32) + bo_ref[...]


@functools.partial(
    pl.kernel,
    out_type=(jax.ShapeDtypeStruct((EPAD, 16), F32),
              jax.ShapeDtypeStruct((EPAD, 16), F32)),
    mesh=_mesh,
    scratch_types=[
        pltpu.VMEM((NCH, CH), jnp.int32),
        pltpu.VMEM((NCH, CH), jnp.int32),
        pltpu.VMEM((CH, 16), F32),
        pltpu.SemaphoreType.DMA,
    ],
)
def _gather2_kernel(h_hbm, src_hbm, tgt_hbm, gs_hbm, gt_hbm,
                    si_v, ti_v, rows_v, sem):
    wid = lax.axis_index("s") * NC + lax.axis_index("c")
    crow = wid * NCH
    pltpu.sync_copy(src_hbm.at[pl.ds(crow, NCH)], si_v)
    pltpu.sync_copy(tgt_hbm.at[pl.ds(crow, NCH)], ti_v)
    ebase = wid * EPW

    def body(k, carry):
        off = ebase + k * CH
        pltpu.async_copy(h_hbm.at[si_v.at[k]], rows_v, sem).wait()
        pltpu.sync_copy(rows_v, gs_hbm.at[pl.ds(off, CH)])
        pltpu.async_copy(h_hbm.at[ti_v.at[k]], rows_v, sem).wait()
        pltpu.sync_copy(rows_v, gt_hbm.at[pl.ds(off, CH)])
        return carry

    lax.fori_loop(0, NCH, body, 0)


@functools.partial(
    pl.kernel,
    out_type=jax.ShapeDtypeStruct((EPAD, 16), F32),
    mesh=_mesh,
    scratch_types=[
        pltpu.VMEM((NCH, CH), jnp.int32),
        pltpu.VMEM((CH, 16), F32),
        pltpu.SemaphoreType.DMA,
    ],
)
def _gather1_kernel(h_hbm, tgt_hbm, gz_hbm, ti_v, rows_v, sem):
    wid = lax.axis_index("s") * NC + lax.axis_index("c")
    crow = wid * NCH
    pltpu.sync_copy(tgt_hbm.at[pl.ds(crow, NCH)], ti_v)
    ebase = wid * EPW

    def body(k, carry):
        off = ebase + k * CH
        pltpu.async_copy(h_hbm.at[ti_v.at[k]], rows_v, sem).wait()
        pltpu.sync_copy(rows_v, gz_hbm.at[pl.ds(off, CH)])
        return carry

    lax.fori_loop(0, NCH, body, 0)


@functools.partial(
    pl.kernel,
    out_type=jax.ShapeDtypeStruct((NC, VP, 16), F32),
    mesh=_mesh,
    scratch_types=[
        pltpu.VMEM((NCH, CH), jnp.int32),
        pltpu.VMEM((CH, 16), F32),
        pltpu.VMEM((ZR, 16), F32),
        pltpu.VMEM_SHARED((VP, 16), F32),
        pltpu.SemaphoreType.DMA,
    ],
)
def _scatter2_kernel(u_hbm, w_hbm, src_hbm, tgt_hbm, out_hbm,
                     idx_v, val_v, stage_v, acc_sh, sem):
    cid = lax.axis_index("c")
    sid = lax.axis_index("s")
    wid = sid * NC + cid

    def zb(i, carry):
        stage_v[i] = jnp.zeros((16,), F32)
        return carry

    lax.fori_loop(0, ZR, zb, 0)
    pltpu.sync_copy(stage_v, acc_sh.at[pl.ds(sid * ZR, ZR)])
    plsc.subcore_barrier()

    ebase = wid * EPW
    nch = jnp.minimum((E_ - ebase) // CH, NCH)
    crow = wid * NCH

    pltpu.sync_copy(src_hbm.at[pl.ds(crow, NCH)], idx_v)

    def bu(k, carry):
        off = ebase + k * CH
        pltpu.sync_copy(u_hbm.at[pl.ds(off, CH)], val_v)
        pltpu.sync_copy(val_v, acc_sh.at[idx_v.at[k]], add=True)
        return carry

    lax.fori_loop(0, nch, bu, 0)

    pltpu.sync_copy(tgt_hbm.at[pl.ds(crow, NCH)], idx_v)

    def bw(k, carry):
        off = ebase + k * CH
        pltpu.sync_copy(w_hbm.at[pl.ds(off, CH)], val_v)
        pltpu.sync_copy(val_v, acc_sh.at[idx_v.at[k]], add=True)
        return carry

    lax.fori_loop(0, nch, bw, 0)
    plsc.subcore_barrier()

    pltpu.sync_copy(acc_sh.at[pl.ds(sid * ZR, ZR)], stage_v)
    pltpu.sync_copy(stage_v, out_hbm.at[cid, pl.ds(sid * ZR, ZR)])


def _h0_call(xp, w16, b16):
    return pl.pallas_call(
        _h0_body,
        out_shape=jax.ShapeDtypeStruct((VP, 16), F32),
    )(xp, w16, b16)


def _maps_mv1_call(gs, gt, w1at, w1bt, w2pt):
    return pl.pallas_call(
        _maps_mv1_body,
        grid=(GRID_E,),
        in_specs=[
            pl.BlockSpec((BLK_E, 16), lambda i: (i, 0)),
            pl.BlockSpec((BLK_E, 16), lambda i: (i, 0)),
            pl.BlockSpec((64, 16), lambda i: (0, 0)),
            pl.BlockSpec((64, 16), lambda i: (0, 0)),
            pl.BlockSpec((64, 64), lambda i: (0, 0)),
        ],
        out_specs=[
            pl.BlockSpec((64, BLK_E), lambda i: (0, i)),
            pl.BlockSpec((BLK_E, 16), lambda i: (i, 0)),
            pl.BlockSpec((BLK_E, 16), lambda i: (i, 0)),
        ],
        out_shape=[
            jax.ShapeDtypeStruct((64, E_), F32),
            jax.ShapeDtypeStruct((E_, 16), F32),
            jax.ShapeDtypeStruct((E_, 16), F32),
        ],
    )(gs, gt, w1at, w1bt, w2pt)


def _mv2_call(mt, gz):
    return pl.pallas_call(
        _mv2_body,
        grid=(GRID_E,),
        in_specs=[
            pl.BlockSpec((64, BLK_E), lambda i: (0, i)),
            pl.BlockSpec((BLK_E, 16), lambda i: (i, 0)),
        ],
        out_specs=[
            pl.BlockSpec((BLK_E, 16), lambda i: (i, 0)),
            pl.BlockSpec((BLK_E, 16), lambda i: (i, 0)),
        ],
        out_shape=[
            jax.ShapeDtypeStruct((E_, 16), F32),
            jax.ShapeDtypeStruct((E_, 16), F32),
        ],
    )(mt, gz)


def _cmb_call(acc, h, wl, bl):
    return pl.pallas_call(
        _cmb_body,
        grid=(GRID_V,),
        in_specs=[
            pl.BlockSpec((NC, BLK_V, 16), lambda i: (0, i, 0)),
            pl.BlockSpec((BLK_V, 16), lambda i: (i, 0)),
            pl.BlockSpec((16, 16), lambda i: (0, 0)),
            pl.BlockSpec((1, 16), lambda i: (0, 0)),
        ],
        out_specs=pl.BlockSpec((BLK_V, 16), lambda i: (i, 0)),
        out_shape=jax.ShapeDtypeStruct((VP, 16), F32),
    )(acc, h, wl, bl)


def _cmb_out_call(acc, h, wl, bl, wo, bo):
    return pl.pallas_call(
        _cmb_out_body,
        grid=(GRID_VO,),
        in_specs=[
            pl.BlockSpec((NC, BLK_VO, 16), lambda i: (0, i, 0)),
            pl.BlockSpec((BLK_VO, 16), lambda i: (i, 0)),
            pl.BlockSpec((16, 16), lambda i: (0, 0)),
            pl.BlockSpec((1, 16), lambda i: (0, 0)),
            pl.BlockSpec((16, 64), lambda i: (0, 0)),
            pl.BlockSpec((1, 64), lambda i: (0, 0)),
        ],
        out_specs=pl.BlockSpec((BLK_VO, 64), lambda i: (i, 0)),
        out_shape=jax.ShapeDtypeStruct((V_, 64), F32),
    )(acc, h, wl, bl, wo, bo)


_PERM = tuple(int((c % 8) * 8 + c // 8) for c in range(64))


def kernel(x, edge_index, W_in, b_in, W1, W2, Ws1, bs1, Ws2, bs2, Wout, bout):
    src = edge_index[0].astype(jnp.int32)
    tgt = edge_index[1].astype(jnp.int32)
    src2 = jnp.pad(src, (0, EPAD - E_)).reshape(NW * NCH, CH)
    tgt2 = jnp.pad(tgt, (0, EPAD - E_)).reshape(NW * NCH, CH)

    W_in16 = jnp.pad(W_in, ((0, 0), (0, 8)))
    b_in16 = jnp.pad(b_in, (0, 8)).reshape(1, 16)
    W1aT = jnp.pad(W1[:8].T, ((0, 0), (0, 8)))
    W1bT = jnp.pad(W1[8:].T, ((0, 0), (0, 8)))
    W2PT = W2[:, np.asarray(_PERM)].T
    Ws1_16 = jnp.pad(Ws1, ((0, 8), (0, 8)))
    bs1_16 = jnp.pad(bs1, (0, 8)).reshape(1, 16)
    Ws2_16 = jnp.pad(Ws2, ((0, 8), (0, 8)))
    bs2_16 = jnp.pad(bs2, (0, 8)).reshape(1, 16)
    Woutp = jnp.pad(Wout, ((0, 8), (0, 0)))
    boutr = bout.reshape(1, 64)
    xp = jnp.pad(x, ((0, VP - V_), (0, 0)))

    h16 = _h0_call(xp, W_in16, b_in16)
    gs, gt = _gather2_kernel(h16, src2, tgt2)
    mt, U1, W1v = _maps_mv1_call(gs, gt, W1aT, W1bT, W2PT)
    acc1 = _scatter2_kernel(U1, W1v, src2, tgt2)
    h1 = _cmb_call(acc1, h16, Ws1_16, bs1_16)
    gz2 = _gather1_kernel(h1, tgt2)
    U2, W2v = _mv2_call(mt, gz2)
    acc2 = _scatter2_kernel(U2, W2v, src2, tgt2)
    return _cmb_out_call(acc2, h1, Ws2_16, bs2_16, Woutp, boutr)


# speedup vs baseline: 22.2938x; 22.2938x over previous
import functools

import numpy as np
import jax
import jax.numpy as jnp
from jax import lax
from jax.experimental import pallas as pl
from jax.experimental.pallas import tpu as pltpu
from jax.experimental.pallas import tpu_sc as plsc

F32 = jnp.float32

NC, NS = 2, 16
NW = NC * NS
V_, E_ = 10000, 320000
VP = 10240
EPAD = 327680
CH = 128
EPW = EPAD // NW
NCH = EPW // CH
ZR = VP // NS
BLK_E = 2560
GRID_E = E_ // BLK_E
BLK_V = 2048
GRID_V = VP // BLK_V
BLK_VO = 2000
GRID_VO = V_ // BLK_VO

@functools.cache
def _sc_mesh():
    return plsc.VectorSubcoreMesh(core_axis_name="c", subcore_axis_name="s",
                                  num_cores=NC, num_subcores=NS)


def _h0_body(x_ref, w_ref, b_ref, o_ref):
    o_ref[...] = (
        jnp.dot(x_ref[...], w_ref[...], preferred_element_type=F32) + b_ref[...]
    )


def _edge_matvec(mT, gz):
    B = gz.shape[0]
    z = jnp.transpose(gz, (1, 0))[:8, :]
    zrep = jnp.broadcast_to(z.reshape(8, 1, B), (8, 8, B)).reshape(64, B)
    u = (mT * zrep).reshape(8, 8, B).sum(axis=0)
    urep = jnp.broadcast_to(u.reshape(1, 8, B), (8, 8, B)).reshape(64, B)
    w = (mT * urep).reshape(8, 8, B).sum(axis=1)
    zpad = jnp.zeros((B, 8), F32)
    U = jnp.concatenate([jnp.transpose(-u, (1, 0)), zpad], axis=1)
    W = jnp.concatenate([jnp.transpose(w, (1, 0)), zpad], axis=1)
    return U, W


def _maps_mv1_body(gs_ref, gt_ref, w1at_ref, w1bt_ref, w2pt_ref,
                   mt_ref, u_ref, w_ref):
    gs = gs_ref[...]
    gt = gt_ref[...]
    hidT = lax.dot_general(w1at_ref[...], gs, (((1,), (1,)), ((), ())),
                           preferred_element_type=F32)
    hidT = hidT + lax.dot_general(w1bt_ref[...], gt, (((1,), (1,)), ((), ())),
                                  preferred_element_type=F32)
    hidT = jnp.tanh(hidT)
    mT = jnp.tanh(lax.dot_general(w2pt_ref[...], hidT, (((1,), (0,)), ((), ())),
                                  preferred_element_type=F32))
    mt_ref[...] = mT
    U, W = _edge_matvec(mT, gt)
    u_ref[...] = U
    w_ref[...] = W


def _mv2_body(mt_ref, gz_ref, u_ref, w_ref):
    U, W = _edge_matvec(mt_ref[...], gz_ref[...])
    u_ref[...] = U
    w_ref[...] = W


def _cmb_body(acc_ref, h_ref, wl_ref, bl_ref, o_ref):
    a = acc_ref[0] + acc_ref[1]
    o_ref[...] = jnp.maximum(
        a + jnp.dot(h_ref[...], wl_ref[...], preferred_element_type=F32)
        + bl_ref[...], 0.0)


def _cmb_out_body(acc_ref, h_ref, wl_ref, bl_ref, wo_ref, bo_ref, o_ref):
    a = acc_ref[0] + acc_ref[1]
    h2 = jnp.maximum(
        a + jnp.dot(h_ref[...], wl_ref[...], preferred_element_type=F32)
        + bl_ref[...], 0.0)
    o_ref[...] = jnp.dot(h2, wo_ref[...], preferred_element_type=F32) + bo_ref[...]


@functools.cache
def _gather2_build():
    return pl.kernel(
        _gather2_body,
        out_type=(jax.ShapeDtypeStruct((EPAD, 16), F32),
                  jax.ShapeDtypeStruct((EPAD, 16), F32)),
        mesh=_sc_mesh(),
        compiler_params=pltpu.CompilerParams(use_tc_tiling_on_sc=False),
        scratch_types=[
            pltpu.VMEM((NCH, CH), jnp.int32),
            pltpu.VMEM((NCH, CH), jnp.int32),
            pltpu.VMEM((CH, 16), F32),
            pltpu.SemaphoreType.DMA,
        ],
    )


def _gather2_body(h_hbm, src_hbm, tgt_hbm, gs_hbm, gt_hbm,
                  si_v, ti_v, rows_v, sem):
    wid = lax.axis_index("s") * NC + lax.axis_index("c")
    crow = wid * NCH
    pltpu.sync_copy(src_hbm.at[pl.ds(crow, NCH)], si_v)
    pltpu.sync_copy(tgt_hbm.at[pl.ds(crow, NCH)], ti_v)
    ebase = wid * EPW

    def body(k, carry):
        off = ebase + k * CH
        pltpu.async_copy(h_hbm.at[si_v.at[k]], rows_v, sem).wait()
        pltpu.sync_copy(rows_v, gs_hbm.at[pl.ds(off, CH)])
        pltpu.async_copy(h_hbm.at[ti_v.at[k]], rows_v, sem).wait()
        pltpu.sync_copy(rows_v, gt_hbm.at[pl.ds(off, CH)])
        return carry

    lax.fori_loop(0, NCH, body, 0)


@functools.cache
def _gather1_build():
    return pl.kernel(
        _gather1_body,
        out_type=jax.ShapeDtypeStruct((EPAD, 16), F32),
        mesh=_sc_mesh(),
        compiler_params=pltpu.CompilerParams(use_tc_tiling_on_sc=False),
        scratch_types=[
            pltpu.VMEM((NCH, CH), jnp.int32),
            pltpu.VMEM((CH, 16), F32),
            pltpu.SemaphoreType.DMA,
        ],
    )


def _gather1_body(h_hbm, tgt_hbm, gz_hbm, ti_v, rows_v, sem):
    wid = lax.axis_index("s") * NC + lax.axis_index("c")
    crow = wid * NCH
    pltpu.sync_copy(tgt_hbm.at[pl.ds(crow, NCH)], ti_v)
    ebase = wid * EPW

    def body(k, carry):
        off = ebase + k * CH
        pltpu.async_copy(h_hbm.at[ti_v.at[k]], rows_v, sem).wait()
        pltpu.sync_copy(rows_v, gz_hbm.at[pl.ds(off, CH)])
        return carry

    lax.fori_loop(0, NCH, body, 0)


@functools.cache
def _scatter2_build():
    return pl.kernel(
        _scatter2_body,
        out_type=jax.ShapeDtypeStruct((NC, VP, 16), F32),
        mesh=_sc_mesh(),
        compiler_params=pltpu.CompilerParams(use_tc_tiling_on_sc=False),
        scratch_types=[
            pltpu.VMEM((NCH, CH), jnp.int32),
            pltpu.VMEM((CH, 16), F32),
            pltpu.VMEM((ZR, 16), F32),
            pltpu.VMEM_SHARED((VP, 16), F32),
            pltpu.SemaphoreType.DMA,
        ],
    )


def _scatter2_body(u_hbm, w_hbm, src_hbm, tgt_hbm, out_hbm,
                   idx_v, val_v, stage_v, acc_sh, sem):
    cid = lax.axis_index("c")
    sid = lax.axis_index("s")
    wid = sid * NC + cid

    def zb(i, carry):
        stage_v[i] = jnp.zeros((16,), F32)
        return carry

    lax.fori_loop(0, ZR, zb, 0)
    pltpu.sync_copy(stage_v, acc_sh.at[pl.ds(sid * ZR, ZR)])
    plsc.subcore_barrier()

    ebase = wid * EPW
    nch = jnp.minimum((E_ - ebase) // CH, NCH)
    crow = wid * NCH

    pltpu.sync_copy(src_hbm.at[pl.ds(crow, NCH)], idx_v)

    def bu(k, carry):
        off = ebase + k * CH
        pltpu.sync_copy(u_hbm.at[pl.ds(off, CH)], val_v)
        pltpu.sync_copy(val_v, acc_sh.at[idx_v.at[k]], add=True)
        return carry

    lax.fori_loop(0, nch, bu, 0)

    pltpu.sync_copy(tgt_hbm.at[pl.ds(crow, NCH)], idx_v)

    def bw(k, carry):
        off = ebase + k * CH
        pltpu.sync_copy(w_hbm.at[pl.ds(off, CH)], val_v)
        pltpu.sync_copy(val_v, acc_sh.at[idx_v.at[k]], add=True)
        return carry

    lax.fori_loop(0, nch, bw, 0)
    plsc.subcore_barrier()

    pltpu.sync_copy(acc_sh.at[pl.ds(sid * ZR, ZR)], stage_v)
    pltpu.sync_copy(stage_v, out_hbm.at[cid, pl.ds(sid * ZR, ZR)])


def _h0_call(xp, w16, b16):
    return pl.pallas_call(
        _h0_body,
        out_shape=jax.ShapeDtypeStruct((VP, 16), F32),
    )(xp, w16, b16)


def _maps_mv1_call(gs, gt, w1at, w1bt, w2pt):
    return pl.pallas_call(
        _maps_mv1_body,
        grid=(GRID_E,),
        in_specs=[
            pl.BlockSpec((BLK_E, 16), lambda i: (i, 0)),
            pl.BlockSpec((BLK_E, 16), lambda i: (i, 0)),
            pl.BlockSpec((64, 16), lambda i: (0, 0)),
            pl.BlockSpec((64, 16), lambda i: (0, 0)),
            pl.BlockSpec((64, 64), lambda i: (0, 0)),
        ],
        out_specs=[
            pl.BlockSpec((64, BLK_E), lambda i: (0, i)),
            pl.BlockSpec((BLK_E, 16), lambda i: (i, 0)),
            pl.BlockSpec((BLK_E, 16), lambda i: (i, 0)),
        ],
        out_shape=[
            jax.ShapeDtypeStruct((64, E_), F32),
            jax.ShapeDtypeStruct((E_, 16), F32),
            jax.ShapeDtypeStruct((E_, 16), F32),
        ],
    )(gs, gt, w1at, w1bt, w2pt)


def _mv2_call(mt, gz):
    return pl.pallas_call(
        _mv2_body,
        grid=(GRID_E,),
        in_specs=[
            pl.BlockSpec((64, BLK_E), lambda i: (0, i)),
            pl.BlockSpec((BLK_E, 16), lambda i: (i, 0)),
        ],
        out_specs=[
            pl.BlockSpec((BLK_E, 16), lambda i: (i, 0)),
            pl.BlockSpec((BLK_E, 16), lambda i: (i, 0)),
        ],
        out_shape=[
            jax.ShapeDtypeStruct((E_, 16), F32),
            jax.ShapeDtypeStruct((E_, 16), F32),
        ],
    )(mt, gz)


def _cmb_call(acc, h, wl, bl):
    return pl.pallas_call(
        _cmb_body,
        grid=(GRID_V,),
        in_specs=[
            pl.BlockSpec((NC, BLK_V, 16), lambda i: (0, i, 0)),
            pl.BlockSpec((BLK_V, 16), lambda i: (i, 0)),
            pl.BlockSpec((16, 16), lambda i: (0, 0)),
            pl.BlockSpec((1, 16), lambda i: (0, 0)),
        ],
        out_specs=pl.BlockSpec((BLK_V, 16), lambda i: (i, 0)),
        out_shape=jax.ShapeDtypeStruct((VP, 16), F32),
    )(acc, h, wl, bl)


def _cmb_out_call(acc, h, wl, bl, wo, bo):
    return pl.pallas_call(
        _cmb_out_body,
        grid=(GRID_VO,),
        in_specs=[
            pl.BlockSpec((NC, BLK_VO, 16), lambda i: (0, i, 0)),
            pl.BlockSpec((BLK_VO, 16), lambda i: (i, 0)),
            pl.BlockSpec((16, 16), lambda i: (0, 0)),
            pl.BlockSpec((1, 16), lambda i: (0, 0)),
            pl.BlockSpec((16, 64), lambda i: (0, 0)),
            pl.BlockSpec((1, 64), lambda i: (0, 0)),
        ],
        out_specs=pl.BlockSpec((BLK_VO, 64), lambda i: (i, 0)),
        out_shape=jax.ShapeDtypeStruct((V_, 64), F32),
    )(acc, h, wl, bl, wo, bo)


_PERM = tuple(int((c % 8) * 8 + c // 8) for c in range(64))


def kernel(x, edge_index, W_in, b_in, W1, W2, Ws1, bs1, Ws2, bs2, Wout, bout):
    src = edge_index[0].astype(jnp.int32)
    tgt = edge_index[1].astype(jnp.int32)
    src2 = jnp.pad(src, (0, EPAD - E_)).reshape(NW * NCH, CH)
    tgt2 = jnp.pad(tgt, (0, EPAD - E_)).reshape(NW * NCH, CH)

    W_in16 = jnp.pad(W_in, ((0, 0), (0, 8)))
    b_in16 = jnp.pad(b_in, (0, 8)).reshape(1, 16)
    W1aT = jnp.pad(W1[:8].T, ((0, 0), (0, 8)))
    W1bT = jnp.pad(W1[8:].T, ((0, 0), (0, 8)))
    W2PT = W2[:, np.asarray(_PERM)].T
    Ws1_16 = jnp.pad(Ws1, ((0, 8), (0, 8)))
    bs1_16 = jnp.pad(bs1, (0, 8)).reshape(1, 16)
    Ws2_16 = jnp.pad(Ws2, ((0, 8), (0, 8)))
    bs2_16 = jnp.pad(bs2, (0, 8)).reshape(1, 16)
    Woutp = jnp.pad(Wout, ((0, 8), (0, 0)))
    boutr = bout.reshape(1, 64)
    xp = jnp.pad(x, ((0, VP - V_), (0, 0)))

    h16 = _h0_call(xp, W_in16, b_in16)
    gs, gt = _gather2_build()(h16, src2, tgt2)
    mt, U1, W1v = _maps_mv1_call(gs, gt, W1aT, W1bT, W2PT)
    acc1 = _scatter2_build()(U1, W1v, src2, tgt2)
    h1 = _cmb_call(acc1, h16, Ws1_16, bs1_16)
    gz2 = _gather1_build()(h1, tgt2)
    U2, W2v = _mv2_call(mt, gz2)
    acc2 = _scatter2_build()(U2, W2v, src2, tgt2)
    return _cmb_out_call(acc2, h1, Ws2_16, bs2_16, Woutp, boutr)


# speedup vs baseline: 46.1562x; 2.0704x over previous
import functools

import numpy as np
import jax
import jax.numpy as jnp
from jax import lax
from jax.experimental import pallas as pl
from jax.experimental.pallas import tpu as pltpu
from jax.experimental.pallas import tpu_sc as plsc

F32 = jnp.float32

NC, NS = 2, 16
NW = NC * NS
V_, E_ = 10000, 320000
VP = 10240
EPAD = 327680
CH = 128
EPW = EPAD // NW
NCH = EPW // CH
ZR = VP // NS
BLK_E = 2560
GRID_E = E_ // BLK_E
BLK_V = 2048
GRID_V = VP // BLK_V


@functools.cache
def _sc_mesh():
    return plsc.VectorSubcoreMesh(core_axis_name="c", subcore_axis_name="s",
                                  num_cores=NC, num_subcores=NS)


def _h0_body(x_ref, w_ref, b_ref, o_ref):
    o_ref[...] = (
        jnp.dot(x_ref[...], w_ref[...], preferred_element_type=F32) + b_ref[...]
    )


def _edge_matvec8(m8, gz8, pz_ref, ru_ref, pu_ref, rw_ref):
    zrep = jnp.dot(gz8, pz_ref[...], preferred_element_type=F32)
    u8 = jnp.dot(m8 * zrep, ru_ref[...], preferred_element_type=F32)
    urep = jnp.dot(u8, pu_ref[...], preferred_element_type=F32)
    w8 = jnp.dot(m8 * urep, rw_ref[...], preferred_element_type=F32)
    return -u8, w8


def _maps_mv1_body(gs_ref, gt_ref, w1a_ref, w1b_ref, w2_ref,
                   pz_ref, ru_ref, pu_ref, rw_ref, mt_ref, u_ref, w_ref):
    gs8 = gs_ref[...]
    gt8 = gt_ref[...]
    hid8 = jnp.tanh(
        jnp.dot(gs8, w1a_ref[...], preferred_element_type=F32)
        + jnp.dot(gt8, w1b_ref[...], preferred_element_type=F32))
    m8 = jnp.tanh(jnp.dot(hid8, w2_ref[...], preferred_element_type=F32))
    mt_ref[...] = m8
    U8, W8 = _edge_matvec8(m8, gt8, pz_ref, ru_ref, pu_ref, rw_ref)
    u_ref[...] = U8
    w_ref[...] = W8


def _mv2_body(mt_ref, gz_ref, pz_ref, ru_ref, pu_ref, rw_ref, u_ref, w_ref):
    U8, W8 = _edge_matvec8(mt_ref[...], gz_ref[...],
                           pz_ref, ru_ref, pu_ref, rw_ref)
    u_ref[...] = U8
    w_ref[...] = W8


def _cmb_body(acc_ref, h_ref, wl_ref, bl_ref, o_ref):
    a = acc_ref[0] + acc_ref[1]
    o_ref[...] = jnp.maximum(
        a + jnp.dot(h_ref[...], wl_ref[...], preferred_element_type=F32)
        + bl_ref[...], 0.0)


def _cmb_out_body(acc_ref, h_ref, wl_ref, bl_ref, wo_ref, bo_ref, o_ref):
    a = acc_ref[0] + acc_ref[1]
    h2 = jnp.maximum(
        a + jnp.dot(h_ref[...], wl_ref[...], preferred_element_type=F32)
        + bl_ref[...], 0.0)
    o_ref[...] = jnp.dot(h2, wo_ref[...], preferred_element_type=F32) + bo_ref[...]


@functools.cache
def _gather2_build():
    return pl.kernel(
        _gather2_body,
        out_type=(jax.ShapeDtypeStruct((EPAD, 16), F32),
                  jax.ShapeDtypeStruct((EPAD, 16), F32)),
        mesh=_sc_mesh(),
        compiler_params=pltpu.CompilerParams(use_tc_tiling_on_sc=False),
        scratch_types=[
            pltpu.VMEM((NCH, CH), jnp.int32),
            pltpu.VMEM((NCH, CH), jnp.int32),
            pltpu.VMEM((CH, 16), F32),
            pltpu.SemaphoreType.DMA,
        ],
    )


def _gather2_body(h_hbm, src_hbm, tgt_hbm, gs_hbm, gt_hbm,
                  si_v, ti_v, rows_v, sem):
    wid = lax.axis_index("s") * NC + lax.axis_index("c")
    crow = wid * NCH
    pltpu.sync_copy(src_hbm.at[pl.ds(crow, NCH)], si_v)
    pltpu.sync_copy(tgt_hbm.at[pl.ds(crow, NCH)], ti_v)
    ebase = wid * EPW

    def body(k, carry):
        off = ebase + k * CH
        pltpu.async_copy(h_hbm.at[si_v.at[k]], rows_v, sem).wait()
        pltpu.sync_copy(rows_v, gs_hbm.at[pl.ds(off, CH)])
        pltpu.async_copy(h_hbm.at[ti_v.at[k]], rows_v, sem).wait()
        pltpu.sync_copy(rows_v, gt_hbm.at[pl.ds(off, CH)])
        return carry

    lax.fori_loop(0, NCH, body, 0)


@functools.cache
def _gather1_build():
    return pl.kernel(
        _gather1_body,
        out_type=jax.ShapeDtypeStruct((EPAD, 16), F32),
        mesh=_sc_mesh(),
        compiler_params=pltpu.CompilerParams(use_tc_tiling_on_sc=False),
        scratch_types=[
            pltpu.VMEM((NCH, CH), jnp.int32),
            pltpu.VMEM((CH, 16), F32),
            pltpu.SemaphoreType.DMA,
        ],
    )


def _gather1_body(h_hbm, tgt_hbm, gz_hbm, ti_v, rows_v, sem):
    wid = lax.axis_index("s") * NC + lax.axis_index("c")
    crow = wid * NCH
    pltpu.sync_copy(tgt_hbm.at[pl.ds(crow, NCH)], ti_v)
    ebase = wid * EPW

    def body(k, carry):
        off = ebase + k * CH
        pltpu.async_copy(h_hbm.at[ti_v.at[k]], rows_v, sem).wait()
        pltpu.sync_copy(rows_v, gz_hbm.at[pl.ds(off, CH)])
        return carry

    lax.fori_loop(0, NCH, body, 0)


@functools.cache
def _scatter2_build():
    return pl.kernel(
        _scatter2_body,
        out_type=jax.ShapeDtypeStruct((NC, VP, 16), F32),
        mesh=_sc_mesh(),
        compiler_params=pltpu.CompilerParams(use_tc_tiling_on_sc=False),
        scratch_types=[
            pltpu.VMEM((NCH, CH), jnp.int32),
            pltpu.VMEM((CH, 16), F32),
            pltpu.VMEM((ZR, 16), F32),
            pltpu.VMEM_SHARED((VP, 16), F32),
            pltpu.SemaphoreType.DMA,
        ],
    )


def _scatter2_body(u_hbm, w_hbm, src_hbm, tgt_hbm, out_hbm,
                   idx_v, val_v, stage_v, acc_sh, sem):
    cid = lax.axis_index("c")
    sid = lax.axis_index("s")
    wid = sid * NC + cid

    def zb(i, carry):
        stage_v[i] = jnp.zeros((16,), F32)
        return carry

    lax.fori_loop(0, ZR, zb, 0)
    pltpu.sync_copy(stage_v, acc_sh.at[pl.ds(sid * ZR, ZR)])
    plsc.subcore_barrier()

    ebase = wid * EPW
    nch = jnp.minimum((E_ - ebase) // CH, NCH)
    crow = wid * NCH

    pltpu.sync_copy(src_hbm.at[pl.ds(crow, NCH)], idx_v)

    def bu(k, carry):
        off = ebase + k * CH
        pltpu.sync_copy(u_hbm.at[pl.ds(off, CH)], val_v)
        pltpu.sync_copy(val_v, acc_sh.at[idx_v.at[k]], add=True)
        return carry

    lax.fori_loop(0, nch, bu, 0)

    pltpu.sync_copy(tgt_hbm.at[pl.ds(crow, NCH)], idx_v)

    def bw(k, carry):
        off = ebase + k * CH
        pltpu.sync_copy(w_hbm.at[pl.ds(off, CH)], val_v)
        pltpu.sync_copy(val_v, acc_sh.at[idx_v.at[k]], add=True)
        return carry

    lax.fori_loop(0, nch, bw, 0)
    plsc.subcore_barrier()

    pltpu.sync_copy(acc_sh.at[pl.ds(sid * ZR, ZR)], stage_v)
    pltpu.sync_copy(stage_v, out_hbm.at[cid, pl.ds(sid * ZR, ZR)])


def _h0_call(xpk, wbd, b8):
    return pl.pallas_call(
        _h0_body,
        out_shape=jax.ShapeDtypeStruct((VP // 8, 128), F32),
    )(xpk, wbd, b8)


def _maps_mv1_call(gs8, gt8, w1a, w1b, w2, pz, ru, pu, rw):
    zero2 = lambda i: (0, 0)
    return pl.pallas_call(
        _maps_mv1_body,
        grid=(GRID_E,),
        in_specs=[
            pl.BlockSpec((BLK_E // 8, 128), lambda i: (i, 0)),
            pl.BlockSpec((BLK_E // 8, 128), lambda i: (i, 0)),
            pl.BlockSpec((128, 512), zero2),
            pl.BlockSpec((128, 512), zero2),
            pl.BlockSpec((512, 512), zero2),
            pl.BlockSpec((128, 512), zero2),
            pl.BlockSpec((512, 128), zero2),
            pl.BlockSpec((128, 512), zero2),
            pl.BlockSpec((512, 128), zero2),
        ],
        out_specs=[
            pl.BlockSpec((BLK_E // 8, 512), lambda i: (i, 0)),
            pl.BlockSpec((BLK_E // 8, 128), lambda i: (i, 0)),
            pl.BlockSpec((BLK_E // 8, 128), lambda i: (i, 0)),
        ],
        out_shape=[
            jax.ShapeDtypeStruct((E_ // 8, 512), F32),
            jax.ShapeDtypeStruct((E_ // 8, 128), F32),
            jax.ShapeDtypeStruct((E_ // 8, 128), F32),
        ],
    )(gs8, gt8, w1a, w1b, w2, pz, ru, pu, rw)


def _mv2_call(mt8, gz8, pz, ru, pu, rw):
    zero2 = lambda i: (0, 0)
    return pl.pallas_call(
        _mv2_body,
        grid=(GRID_E,),
        in_specs=[
            pl.BlockSpec((BLK_E // 8, 512), lambda i: (i, 0)),
            pl.BlockSpec((BLK_E // 8, 128), lambda i: (i, 0)),
            pl.BlockSpec((128, 512), zero2),
            pl.BlockSpec((512, 128), zero2),
            pl.BlockSpec((128, 512), zero2),
            pl.BlockSpec((512, 128), zero2),
        ],
        out_specs=[
            pl.BlockSpec((BLK_E // 8, 128), lambda i: (i, 0)),
            pl.BlockSpec((BLK_E // 8, 128), lambda i: (i, 0)),
        ],
        out_shape=[
            jax.ShapeDtypeStruct((E_ // 8, 128), F32),
            jax.ShapeDtypeStruct((E_ // 8, 128), F32),
        ],
    )(mt8, gz8, pz, ru, pu, rw)


def _cmb_call(acc8, h8, wl, bl):
    return pl.pallas_call(
        _cmb_body,
        grid=(GRID_V,),
        in_specs=[
            pl.BlockSpec((NC, BLK_V // 8, 128), lambda i: (0, i, 0)),
            pl.BlockSpec((BLK_V // 8, 128), lambda i: (i, 0)),
            pl.BlockSpec((128, 128), lambda i: (0, 0)),
            pl.BlockSpec((1, 128), lambda i: (0, 0)),
        ],
        out_specs=pl.BlockSpec((BLK_V // 8, 128), lambda i: (i, 0)),
        out_shape=jax.ShapeDtypeStruct((VP // 8, 128), F32),
    )(acc8, h8, wl, bl)


def _cmb_out_call(acc8, h8, wl, bl, wo, bo):
    return pl.pallas_call(
        _cmb_out_body,
        grid=(GRID_V,),
        in_specs=[
            pl.BlockSpec((NC, BLK_V // 8, 128), lambda i: (0, i, 0)),
            pl.BlockSpec((BLK_V // 8, 128), lambda i: (i, 0)),
            pl.BlockSpec((128, 128), lambda i: (0, 0)),
            pl.BlockSpec((1, 128), lambda i: (0, 0)),
            pl.BlockSpec((128, 512), lambda i: (0, 0)),
            pl.BlockSpec((1, 512), lambda i: (0, 0)),
        ],
        out_specs=pl.BlockSpec((BLK_V // 8, 512), lambda i: (i, 0)),
        out_shape=jax.ShapeDtypeStruct((VP // 8, 512), F32),
    )(acc8, h8, wl, bl, wo, bo)


_PERM = np.array([(c % 8) * 8 + c // 8 for c in range(64)])

_PZ = np.zeros((16, 64), np.float32)
_RU = np.zeros((64, 16), np.float32)
_PU = np.zeros((16, 64), np.float32)
_RW = np.zeros((64, 16), np.float32)
for _j in range(8):
    for _i in range(8):
        _PZ[_j, 8 * _j + _i] = 1.0
        _RU[8 * _j + _i, _i] = 1.0
        _PU[_i, 8 * _j + _i] = 1.0
        _RW[8 * _j + _i, _j] = 1.0
_I8 = np.eye(8, dtype=np.float32)
_PZ8 = np.kron(_I8, _PZ)
_RU8 = np.kron(_I8, _RU)
_PU8 = np.kron(_I8, _PU)
_RW8 = np.kron(_I8, _RW)


def kernel(x, edge_index, W_in, b_in, W1, W2, Ws1, bs1, Ws2, bs2, Wout, bout):
    src = edge_index[0].astype(jnp.int32)
    tgt = edge_index[1].astype(jnp.int32)
    src2 = jnp.pad(src, (0, EPAD - E_)).reshape(NW * NCH, CH)
    tgt2 = jnp.pad(tgt, (0, EPAD - E_)).reshape(NW * NCH, CH)

    i8 = jnp.asarray(_I8)
    W_in16 = jnp.pad(W_in, ((0, 0), (0, 8)))
    W_in_bd = jnp.kron(i8, W_in16)
    b_in8 = jnp.tile(jnp.pad(b_in, (0, 8)).reshape(1, 16), (1, 8))
    W1a16 = jnp.pad(W1[:8], ((0, 8), (0, 0)))
    W1b16 = jnp.pad(W1[8:], ((0, 8), (0, 0)))
    W1A = jnp.kron(i8, W1a16)
    W1B = jnp.kron(i8, W1b16)
    W2BD = jnp.kron(i8, W2[:, _PERM])
    WL1 = jnp.kron(i8, jnp.pad(Ws1, ((0, 8), (0, 8))))
    WL2 = jnp.kron(i8, jnp.pad(Ws2, ((0, 8), (0, 8))))
    bl1_8 = jnp.tile(jnp.pad(bs1, (0, 8)).reshape(1, 16), (1, 8))
    bl2_8 = jnp.tile(jnp.pad(bs2, (0, 8)).reshape(1, 16), (1, 8))
    WOBD = jnp.kron(i8, jnp.pad(Wout, ((0, 8), (0, 0))))
    bo8 = jnp.tile(bout.reshape(1, 64), (1, 8))
    xpk = jnp.pad(x, ((0, VP - V_), (0, 0))).reshape(VP // 8, 1024)

    h8 = _h0_call(xpk, W_in_bd, b_in8)
    h16 = jnp.reshape(h8, (VP, 16))
    gs, gt = _gather2_build()(h16, src2, tgt2)
    gs8 = jnp.reshape(gs, (EPAD // 8, 128))
    gt8 = jnp.reshape(gt, (EPAD // 8, 128))
    pz8, ru8, pu8, rw8 = (jnp.asarray(_PZ8), jnp.asarray(_RU8),
                          jnp.asarray(_PU8), jnp.asarray(_RW8))
    mt8, U1, W1v = _maps_mv1_call(gs8, gt8, W1A, W1B, W2BD,
                                  pz8, ru8, pu8, rw8)
    acc1 = _scatter2_build()(jnp.reshape(U1, (E_, 16)),
                             jnp.reshape(W1v, (E_, 16)), src2, tgt2)
    acc1_8 = jnp.reshape(acc1, (NC, VP // 8, 128))
    h1_8 = _cmb_call(acc1_8, h8, WL1, bl1_8)
    h1 = jnp.reshape(h1_8, (VP, 16))
    gz2 = _gather1_build()(h1, tgt2)
    gz2_8 = jnp.reshape(gz2, (EPAD // 8, 128))
    U2, W2v = _mv2_call(mt8, gz2_8, pz8, ru8, pu8, rw8)
    acc2 = _scatter2_build()(jnp.reshape(U2, (E_, 16)),
                             jnp.reshape(W2v, (E_, 16)), src2, tgt2)
    acc2_8 = jnp.reshape(acc2, (NC, VP // 8, 128))
    outp = _cmb_out_call(acc2_8, h1_8, WL2, bl2_8, WOBD, bo8)
    return jnp.reshape(outp, (VP, 64))[:V_]


# speedup vs baseline: 69.9477x; 1.5155x over previous
import functools

import numpy as np
import jax
import jax.numpy as jnp
from jax import lax
from jax.experimental import pallas as pl
from jax.experimental.pallas import tpu as pltpu
from jax.experimental.pallas import tpu_sc as plsc

F32 = jnp.float32

NC, NS = 2, 16
NW = NC * NS
V_, E_ = 10000, 320000
VP = 10240
EPAD = 327680
CH = 128
EPW = EPAD // NW
NCH = EPW // CH
ZR = VP // NS
BLK_E = 2560
GRID_E = E_ // BLK_E
BLK_V = 2048
GRID_V = VP // BLK_V


@functools.cache
def _sc_mesh():
    return plsc.VectorSubcoreMesh(core_axis_name="c", subcore_axis_name="s",
                                  num_cores=NC, num_subcores=NS)


def _h0_body(x_ref, w_ref, b_ref, o_ref):
    o_ref[...] = (
        jnp.dot(x_ref[...], w_ref[...], preferred_element_type=F32) + b_ref[...]
    )


def _edge_matvec8(m8, gz8, pz_ref, ru_ref, pu_ref, rw_ref):
    zrep = jnp.dot(gz8, pz_ref[...], preferred_element_type=F32)
    u8 = jnp.dot(m8 * zrep, ru_ref[...], preferred_element_type=F32)
    urep = jnp.dot(u8, pu_ref[...], preferred_element_type=F32)
    w8 = jnp.dot(m8 * urep, rw_ref[...], preferred_element_type=F32)
    return -u8, w8


def _maps_mv1_body(gs_ref, gt_ref, w1a_ref, w1b_ref, w2_ref,
                   pz_ref, ru_ref, pu_ref, rw_ref, mt_ref, u_ref, w_ref):
    gs8 = gs_ref[...]
    gt8 = gt_ref[...]
    hid8 = jnp.tanh(
        jnp.dot(gs8, w1a_ref[...], preferred_element_type=F32)
        + jnp.dot(gt8, w1b_ref[...], preferred_element_type=F32))
    m8 = jnp.tanh(jnp.dot(hid8, w2_ref[...], preferred_element_type=F32))
    mt_ref[...] = m8
    U8, W8 = _edge_matvec8(m8, gt8, pz_ref, ru_ref, pu_ref, rw_ref)
    u_ref[...] = U8
    w_ref[...] = W8


def _mv2_body(mt_ref, gz_ref, pz_ref, ru_ref, pu_ref, rw_ref, u_ref, w_ref):
    U8, W8 = _edge_matvec8(mt_ref[...], gz_ref[...],
                           pz_ref, ru_ref, pu_ref, rw_ref)
    u_ref[...] = U8
    w_ref[...] = W8


def _cmb_body(acc_ref, h_ref, wl_ref, bl_ref, o_ref):
    a = acc_ref[0] + acc_ref[1]
    o_ref[...] = jnp.maximum(
        a + jnp.dot(h_ref[...], wl_ref[...], preferred_element_type=F32)
        + bl_ref[...], 0.0)


def _cmb_out_body(acc_ref, h_ref, wl_ref, bl_ref, wo_ref, bo_ref, o_ref):
    a = acc_ref[0] + acc_ref[1]
    h2 = jnp.maximum(
        a + jnp.dot(h_ref[...], wl_ref[...], preferred_element_type=F32)
        + bl_ref[...], 0.0)
    o_ref[...] = jnp.dot(h2, wo_ref[...], preferred_element_type=F32) + bo_ref[...]


NB = 8


def _gather_stream(h_hbm, idx_v, out_hbm, rows_v, gsem, osem, ebase):
    for b in range(NB):
        pltpu.async_copy(h_hbm.at[idx_v.at[b]], rows_v.at[b], gsem.at[b])

    def body(k, carry):
        @pl.when(k > 0)
        def _():
            pc = k - 1 + NB

            @pl.when(pc < NCH)
            def _():
                pb = lax.rem(k - 1, NB)
                pltpu.make_async_copy(
                    rows_v.at[pb],
                    out_hbm.at[pl.ds(ebase + (k - 1) * CH, CH)],
                    osem.at[pb]).wait()
                pltpu.async_copy(h_hbm.at[idx_v.at[pc]], rows_v.at[pb],
                                 gsem.at[pb])

        b = lax.rem(k, NB)
        pltpu.make_async_copy(h_hbm.at[idx_v.at[k]], rows_v.at[b],
                              gsem.at[b]).wait()
        pltpu.async_copy(rows_v.at[b], out_hbm.at[pl.ds(ebase + k * CH, CH)],
                         osem.at[b])
        return carry

    lax.fori_loop(0, NCH, body, 0)

    def drain(b, carry):
        c = NCH - NB + b
        pltpu.make_async_copy(rows_v.at[b],
                              out_hbm.at[pl.ds(ebase + c * CH, CH)],
                              osem.at[b]).wait()
        return carry

    lax.fori_loop(0, NB, drain, 0)


@functools.cache
def _gather2_build():
    return pl.kernel(
        _gather2_body,
        out_type=(jax.ShapeDtypeStruct((EPAD, 16), F32),
                  jax.ShapeDtypeStruct((EPAD, 16), F32)),
        mesh=_sc_mesh(),
        compiler_params=pltpu.CompilerParams(use_tc_tiling_on_sc=False),
        scratch_types=[
            pltpu.VMEM((NCH, CH), jnp.int32),
            pltpu.VMEM((NCH, CH), jnp.int32),
            pltpu.VMEM((NB, CH, 16), F32),
            pltpu.SemaphoreType.DMA((NB,)),
            pltpu.SemaphoreType.DMA((NB,)),
        ],
    )


def _gather2_body(h_hbm, src_hbm, tgt_hbm, gs_hbm, gt_hbm,
                  si_v, ti_v, rows_v, gsem, osem):
    wid = lax.axis_index("s") * NC + lax.axis_index("c")
    crow = wid * NCH
    pltpu.sync_copy(src_hbm.at[pl.ds(crow, NCH)], si_v)
    pltpu.sync_copy(tgt_hbm.at[pl.ds(crow, NCH)], ti_v)
    ebase = wid * EPW
    _gather_stream(h_hbm, si_v, gs_hbm, rows_v, gsem, osem, ebase)
    _gather_stream(h_hbm, ti_v, gt_hbm, rows_v, gsem, osem, ebase)


@functools.cache
def _gather1_build():
    return pl.kernel(
        _gather1_body,
        out_type=jax.ShapeDtypeStruct((EPAD, 16), F32),
        mesh=_sc_mesh(),
        compiler_params=pltpu.CompilerParams(use_tc_tiling_on_sc=False),
        scratch_types=[
            pltpu.VMEM((NCH, CH), jnp.int32),
            pltpu.VMEM((NB, CH, 16), F32),
            pltpu.SemaphoreType.DMA((NB,)),
            pltpu.SemaphoreType.DMA((NB,)),
        ],
    )


def _gather1_body(h_hbm, tgt_hbm, gz_hbm, ti_v, rows_v, gsem, osem):
    wid = lax.axis_index("s") * NC + lax.axis_index("c")
    crow = wid * NCH
    pltpu.sync_copy(tgt_hbm.at[pl.ds(crow, NCH)], ti_v)
    ebase = wid * EPW
    _gather_stream(h_hbm, ti_v, gz_hbm, rows_v, gsem, osem, ebase)


def _scatter_stream(val_hbm, idx_v, acc_sh, vbuf, lsem, ssem, ebase, nch):
    for b in range(NB):
        pltpu.async_copy(val_hbm.at[pl.ds(ebase + b * CH, CH)], vbuf.at[b],
                         lsem.at[b])

    def body(k, carry):
        @pl.when(k > 0)
        def _():
            pc = k - 1 + NB

            @pl.when(pc < nch)
            def _():
                pb = lax.rem(k - 1, NB)
                pltpu.make_async_copy(vbuf.at[pb],
                                      acc_sh.at[idx_v.at[k - 1]],
                                      ssem.at[pb]).wait()
                pltpu.async_copy(val_hbm.at[pl.ds(ebase + pc * CH, CH)],
                                 vbuf.at[pb], lsem.at[pb])

        b = lax.rem(k, NB)
        pltpu.make_async_copy(val_hbm.at[pl.ds(ebase + k * CH, CH)],
                              vbuf.at[b], lsem.at[b]).wait()
        pltpu.async_copy(vbuf.at[b], acc_sh.at[idx_v.at[k]], ssem.at[b],
                         add=True)
        return carry

    lax.fori_loop(0, nch, body, 0)

    def drain(b, carry):
        c = nch - NB + b
        cb = lax.rem(c, NB)
        pltpu.make_async_copy(vbuf.at[cb], acc_sh.at[idx_v.at[c]],
                              ssem.at[cb]).wait()
        return carry

    lax.fori_loop(0, NB, drain, 0)


@functools.cache
def _scatter2_build():
    return pl.kernel(
        _scatter2_body,
        out_type=jax.ShapeDtypeStruct((NC, VP, 16), F32),
        mesh=_sc_mesh(),
        compiler_params=pltpu.CompilerParams(use_tc_tiling_on_sc=False),
        scratch_types=[
            pltpu.VMEM((NCH, CH), jnp.int32),
            pltpu.VMEM((NB, CH, 16), F32),
            pltpu.VMEM((ZR, 16), F32),
            pltpu.VMEM_SHARED((VP, 16), F32),
            pltpu.SemaphoreType.DMA((NB,)),
            pltpu.SemaphoreType.DMA((NB,)),
        ],
    )


def _scatter2_body(u_hbm, w_hbm, src_hbm, tgt_hbm, out_hbm,
                   idx_v, vbuf, stage_v, acc_sh, lsem, ssem):
    cid = lax.axis_index("c")
    sid = lax.axis_index("s")
    wid = sid * NC + cid

    def zb(i, carry):
        stage_v[i] = jnp.zeros((16,), F32)
        return carry

    lax.fori_loop(0, ZR, zb, 0)
    pltpu.sync_copy(stage_v, acc_sh.at[pl.ds(sid * ZR, ZR)])
    plsc.subcore_barrier()

    ebase = wid * EPW
    nch = jnp.minimum((E_ - ebase) // CH, NCH)
    crow = wid * NCH

    pltpu.sync_copy(src_hbm.at[pl.ds(crow, NCH)], idx_v)
    _scatter_stream(u_hbm, idx_v, acc_sh, vbuf, lsem, ssem, ebase, nch)
    pltpu.sync_copy(tgt_hbm.at[pl.ds(crow, NCH)], idx_v)
    _scatter_stream(w_hbm, idx_v, acc_sh, vbuf, lsem, ssem, ebase, nch)

    plsc.subcore_barrier()
    pltpu.sync_copy(acc_sh.at[pl.ds(sid * ZR, ZR)], stage_v)
    pltpu.sync_copy(stage_v, out_hbm.at[cid, pl.ds(sid * ZR, ZR)])


def _h0_call(xpk, wbd, b8):
    return pl.pallas_call(
        _h0_body,
        out_shape=jax.ShapeDtypeStruct((VP // 8, 128), F32),
    )(xpk, wbd, b8)


def _maps_mv1_call(gs8, gt8, w1a, w1b, w2, pz, ru, pu, rw):
    zero2 = lambda i: (0, 0)
    return pl.pallas_call(
        _maps_mv1_body,
        grid=(GRID_E,),
        in_specs=[
            pl.BlockSpec((BLK_E // 8, 128), lambda i: (i, 0)),
            pl.BlockSpec((BLK_E // 8, 128), lambda i: (i, 0)),
            pl.BlockSpec((128, 512), zero2),
            pl.BlockSpec((128, 512), zero2),
            pl.BlockSpec((512, 512), zero2),
            pl.BlockSpec((128, 512), zero2),
            pl.BlockSpec((512, 128), zero2),
            pl.BlockSpec((128, 512), zero2),
            pl.BlockSpec((512, 128), zero2),
        ],
        out_specs=[
            pl.BlockSpec((BLK_E // 8, 512), lambda i: (i, 0)),
            pl.BlockSpec((BLK_E // 8, 128), lambda i: (i, 0)),
            pl.BlockSpec((BLK_E // 8, 128), lambda i: (i, 0)),
        ],
        out_shape=[
            jax.ShapeDtypeStruct((E_ // 8, 512), F32),
            jax.ShapeDtypeStruct((E_ // 8, 128), F32),
            jax.ShapeDtypeStruct((E_ // 8, 128), F32),
        ],
    )(gs8, gt8, w1a, w1b, w2, pz, ru, pu, rw)


def _mv2_call(mt8, gz8, pz, ru, pu, rw):
    zero2 = lambda i: (0, 0)
    return pl.pallas_call(
        _mv2_body,
        grid=(GRID_E,),
        in_specs=[
            pl.BlockSpec((BLK_E // 8, 512), lambda i: (i, 0)),
            pl.BlockSpec((BLK_E // 8, 128), lambda i: (i, 0)),
            pl.BlockSpec((128, 512), zero2),
            pl.BlockSpec((512, 128), zero2),
            pl.BlockSpec((128, 512), zero2),
            pl.BlockSpec((512, 128), zero2),
        ],
        out_specs=[
            pl.BlockSpec((BLK_E // 8, 128), lambda i: (i, 0)),
            pl.BlockSpec((BLK_E // 8, 128), lambda i: (i, 0)),
        ],
        out_shape=[
            jax.ShapeDtypeStruct((E_ // 8, 128), F32),
            jax.ShapeDtypeStruct((E_ // 8, 128), F32),
        ],
    )(mt8, gz8, pz, ru, pu, rw)


def _cmb_call(acc8, h8, wl, bl):
    return pl.pallas_call(
        _cmb_body,
        grid=(GRID_V,),
        in_specs=[
            pl.BlockSpec((NC, BLK_V // 8, 128), lambda i: (0, i, 0)),
            pl.BlockSpec((BLK_V // 8, 128), lambda i: (i, 0)),
            pl.BlockSpec((128, 128), lambda i: (0, 0)),
            pl.BlockSpec((1, 128), lambda i: (0, 0)),
        ],
        out_specs=pl.BlockSpec((BLK_V // 8, 128), lambda i: (i, 0)),
        out_shape=jax.ShapeDtypeStruct((VP // 8, 128), F32),
    )(acc8, h8, wl, bl)


def _cmb_out_call(acc8, h8, wl, bl, wo, bo):
    return pl.pallas_call(
        _cmb_out_body,
        grid=(GRID_V,),
        in_specs=[
            pl.BlockSpec((NC, BLK_V // 8, 128), lambda i: (0, i, 0)),
            pl.BlockSpec((BLK_V // 8, 128), lambda i: (i, 0)),
            pl.BlockSpec((128, 128), lambda i: (0, 0)),
            pl.BlockSpec((1, 128), lambda i: (0, 0)),
            pl.BlockSpec((128, 512), lambda i: (0, 0)),
            pl.BlockSpec((1, 512), lambda i: (0, 0)),
        ],
        out_specs=pl.BlockSpec((BLK_V // 8, 512), lambda i: (i, 0)),
        out_shape=jax.ShapeDtypeStruct((VP // 8, 512), F32),
    )(acc8, h8, wl, bl, wo, bo)


_PERM = np.array([(c % 8) * 8 + c // 8 for c in range(64)])

_PZ = np.zeros((16, 64), np.float32)
_RU = np.zeros((64, 16), np.float32)
_PU = np.zeros((16, 64), np.float32)
_RW = np.zeros((64, 16), np.float32)
for _j in range(8):
    for _i in range(8):
        _PZ[_j, 8 * _j + _i] = 1.0
        _RU[8 * _j + _i, _i] = 1.0
        _PU[_i, 8 * _j + _i] = 1.0
        _RW[8 * _j + _i, _j] = 1.0
_I8 = np.eye(8, dtype=np.float32)
_PZ8 = np.kron(_I8, _PZ)
_RU8 = np.kron(_I8, _RU)
_PU8 = np.kron(_I8, _PU)
_RW8 = np.kron(_I8, _RW)


def kernel(x, edge_index, W_in, b_in, W1, W2, Ws1, bs1, Ws2, bs2, Wout, bout):
    src = edge_index[0].astype(jnp.int32)
    tgt = edge_index[1].astype(jnp.int32)
    src2 = jnp.pad(src, (0, EPAD - E_)).reshape(NW * NCH, CH)
    tgt2 = jnp.pad(tgt, (0, EPAD - E_)).reshape(NW * NCH, CH)

    i8 = jnp.asarray(_I8)
    W_in16 = jnp.pad(W_in, ((0, 0), (0, 8)))
    W_in_bd = jnp.kron(i8, W_in16)
    b_in8 = jnp.tile(jnp.pad(b_in, (0, 8)).reshape(1, 16), (1, 8))
    W1a16 = jnp.pad(W1[:8], ((0, 8), (0, 0)))
    W1b16 = jnp.pad(W1[8:], ((0, 8), (0, 0)))
    W1A = jnp.kron(i8, W1a16)
    W1B = jnp.kron(i8, W1b16)
    W2BD = jnp.kron(i8, W2[:, _PERM])
    WL1 = jnp.kron(i8, jnp.pad(Ws1, ((0, 8), (0, 8))))
    WL2 = jnp.kron(i8, jnp.pad(Ws2, ((0, 8), (0, 8))))
    bl1_8 = jnp.tile(jnp.pad(bs1, (0, 8)).reshape(1, 16), (1, 8))
    bl2_8 = jnp.tile(jnp.pad(bs2, (0, 8)).reshape(1, 16), (1, 8))
    WOBD = jnp.kron(i8, jnp.pad(Wout, ((0, 8), (0, 0))))
    bo8 = jnp.tile(bout.reshape(1, 64), (1, 8))
    xpk = jnp.pad(x, ((0, VP - V_), (0, 0))).reshape(VP // 8, 1024)

    h8 = _h0_call(xpk, W_in_bd, b_in8)
    h16 = jnp.reshape(h8, (VP, 16))
    gs, gt = _gather2_build()(h16, src2, tgt2)
    gs8 = jnp.reshape(gs, (EPAD // 8, 128))
    gt8 = jnp.reshape(gt, (EPAD // 8, 128))
    pz8, ru8, pu8, rw8 = (jnp.asarray(_PZ8), jnp.asarray(_RU8),
                          jnp.asarray(_PU8), jnp.asarray(_RW8))
    mt8, U1, W1v = _maps_mv1_call(gs8, gt8, W1A, W1B, W2BD,
                                  pz8, ru8, pu8, rw8)
    acc1 = _scatter2_build()(jnp.reshape(U1, (E_, 16)),
                             jnp.reshape(W1v, (E_, 16)), src2, tgt2)
    acc1_8 = jnp.reshape(acc1, (NC, VP // 8, 128))
    h1_8 = _cmb_call(acc1_8, h8, WL1, bl1_8)
    h1 = jnp.reshape(h1_8, (VP, 16))
    gz2 = _gather1_build()(h1, tgt2)
    gz2_8 = jnp.reshape(gz2, (EPAD // 8, 128))
    U2, W2v = _mv2_call(mt8, gz2_8, pz8, ru8, pu8, rw8)
    acc2 = _scatter2_build()(jnp.reshape(U2, (E_, 16)),
                             jnp.reshape(W2v, (E_, 16)), src2, tgt2)
    acc2_8 = jnp.reshape(acc2, (NC, VP // 8, 128))
    outp = _cmb_out_call(acc2_8, h1_8, WL2, bl2_8, WOBD, bo8)
    return jnp.reshape(outp, (VP, 64))[:V_]


# speedup vs baseline: 93.4350x; 1.3358x over previous
import functools

import numpy as np
import jax
import jax.numpy as jnp
from jax import lax
from jax.experimental import pallas as pl
from jax.experimental.pallas import tpu as pltpu
from jax.experimental.pallas import tpu_sc as plsc

F32 = jnp.float32

NC, NS = 2, 16
NW = NC * NS
V_, E_ = 10000, 320000
VP = 10240
EPAD = 327680
CH = 128
EPW = EPAD // NW
NCH = EPW // CH
ZR = VP // NS
BLK_E = 2560
GRID_E = E_ // BLK_E
BLK_V = 2048
GRID_V = VP // BLK_V


@functools.cache
def _sc_mesh():
    return plsc.VectorSubcoreMesh(core_axis_name="c", subcore_axis_name="s",
                                  num_cores=NC, num_subcores=NS)


def _h0_body(x_ref, w_ref, b_ref, o_ref):
    o_ref[...] = (
        jnp.dot(x_ref[...], w_ref[...], preferred_element_type=F32) + b_ref[...]
    )


def _edge_matvec8(m8, gz8, pz_ref, ru_ref, pu_ref, rw_ref):
    zrep = jnp.dot(gz8, pz_ref[...], preferred_element_type=F32)
    u8 = jnp.dot(m8 * zrep, ru_ref[...], preferred_element_type=F32)
    urep = jnp.dot(u8, pu_ref[...], preferred_element_type=F32)
    w8 = jnp.dot(m8 * urep, rw_ref[...], preferred_element_type=F32)
    return -u8, w8


def _maps_mv1_body(gs_ref, gt_ref, w1a_ref, w1b_ref, w2_ref,
                   pz_ref, ru_ref, pu_ref, rw_ref, mt_ref, u_ref, w_ref):
    gs8 = gs_ref[...]
    gt8 = gt_ref[...]
    hid8 = jnp.tanh(
        jnp.dot(gs8, w1a_ref[...], preferred_element_type=F32)
        + jnp.dot(gt8, w1b_ref[...], preferred_element_type=F32))
    m8 = jnp.tanh(jnp.dot(hid8, w2_ref[...], preferred_element_type=F32))
    mt_ref[...] = m8
    U8, W8 = _edge_matvec8(m8, gt8, pz_ref, ru_ref, pu_ref, rw_ref)
    u_ref[...] = U8
    w_ref[...] = W8


def _mv2_body(mt_ref, gz_ref, pz_ref, ru_ref, pu_ref, rw_ref, u_ref, w_ref):
    U8, W8 = _edge_matvec8(mt_ref[...], gz_ref[...],
                           pz_ref, ru_ref, pu_ref, rw_ref)
    u_ref[...] = U8
    w_ref[...] = W8


def _cmb_body(acc_ref, h_ref, wl_ref, bl_ref, o_ref):
    a = acc_ref[0] + acc_ref[1]
    o_ref[...] = jnp.maximum(
        a + jnp.dot(h_ref[...], wl_ref[...], preferred_element_type=F32)
        + bl_ref[...], 0.0)


def _cmb_out_body(acc_ref, h_ref, wl_ref, bl_ref, wo_ref, bo_ref, o_ref):
    a = acc_ref[0] + acc_ref[1]
    h2 = jnp.maximum(
        a + jnp.dot(h_ref[...], wl_ref[...], preferred_element_type=F32)
        + bl_ref[...], 0.0)
    o_ref[...] = jnp.dot(h2, wo_ref[...], preferred_element_type=F32) + bo_ref[...]


NB = 8


def _gather_stream(h_hbm, idx_v, out_hbm, rows_v, gsem, osem, ebase):
    for b in range(NB):
        pltpu.async_copy(h_hbm.at[idx_v.at[b]], rows_v.at[b], gsem.at[b])

    def body(k, carry):
        @pl.when(k > 0)
        def _():
            pc = k - 1 + NB

            @pl.when(pc < NCH)
            def _():
                pb = lax.rem(k - 1, NB)
                pltpu.make_async_copy(
                    rows_v.at[pb],
                    out_hbm.at[pl.ds(ebase + (k - 1) * CH, CH)],
                    osem.at[pb]).wait()
                pltpu.async_copy(h_hbm.at[idx_v.at[pc]], rows_v.at[pb],
                                 gsem.at[pb])

        b = lax.rem(k, NB)
        pltpu.make_async_copy(h_hbm.at[idx_v.at[k]], rows_v.at[b],
                              gsem.at[b]).wait()
        pltpu.async_copy(rows_v.at[b], out_hbm.at[pl.ds(ebase + k * CH, CH)],
                         osem.at[b])
        return carry

    lax.fori_loop(0, NCH, body, 0)

    def drain(b, carry):
        c = NCH - NB + b
        pltpu.make_async_copy(rows_v.at[b],
                              out_hbm.at[pl.ds(ebase + c * CH, CH)],
                              osem.at[b]).wait()
        return carry

    lax.fori_loop(0, NB, drain, 0)


@functools.cache
def _gather2_build():
    return pl.kernel(
        _gather2_body,
        out_type=(jax.ShapeDtypeStruct((EPAD, 16), F32),
                  jax.ShapeDtypeStruct((EPAD, 16), F32)),
        mesh=_sc_mesh(),
        compiler_params=pltpu.CompilerParams(use_tc_tiling_on_sc=False),
        scratch_types=[
            pltpu.VMEM((NCH, CH), jnp.int32),
            pltpu.VMEM((NCH, CH), jnp.int32),
            pltpu.VMEM((NB, CH, 16), F32),
            pltpu.VMEM_SHARED((VP, 16), F32),
            pltpu.SemaphoreType.DMA((NB,)),
            pltpu.SemaphoreType.DMA((NB,)),
        ],
    )


def _gather2_body(h_hbm, src_hbm, tgt_hbm, gs_hbm, gt_hbm,
                  si_v, ti_v, rows_v, h_sh, gsem, osem):
    sid = lax.axis_index("s")
    wid = sid * NC + lax.axis_index("c")
    crow = wid * NCH
    pltpu.sync_copy(h_hbm.at[pl.ds(sid * ZR, ZR)], h_sh.at[pl.ds(sid * ZR, ZR)])
    pltpu.sync_copy(src_hbm.at[pl.ds(crow, NCH)], si_v)
    pltpu.sync_copy(tgt_hbm.at[pl.ds(crow, NCH)], ti_v)
    plsc.subcore_barrier()
    ebase = wid * EPW
    _gather_stream(h_sh, si_v, gs_hbm, rows_v, gsem, osem, ebase)
    _gather_stream(h_sh, ti_v, gt_hbm, rows_v, gsem, osem, ebase)


@functools.cache
def _gather1_build():
    return pl.kernel(
        _gather1_body,
        out_type=jax.ShapeDtypeStruct((EPAD, 16), F32),
        mesh=_sc_mesh(),
        compiler_params=pltpu.CompilerParams(use_tc_tiling_on_sc=False),
        scratch_types=[
            pltpu.VMEM((NCH, CH), jnp.int32),
            pltpu.VMEM((NB, CH, 16), F32),
            pltpu.VMEM_SHARED((VP, 16), F32),
            pltpu.SemaphoreType.DMA((NB,)),
            pltpu.SemaphoreType.DMA((NB,)),
        ],
    )


def _gather1_body(h_hbm, tgt_hbm, gz_hbm, ti_v, rows_v, h_sh, gsem, osem):
    sid = lax.axis_index("s")
    wid = sid * NC + lax.axis_index("c")
    crow = wid * NCH
    pltpu.sync_copy(h_hbm.at[pl.ds(sid * ZR, ZR)], h_sh.at[pl.ds(sid * ZR, ZR)])
    pltpu.sync_copy(tgt_hbm.at[pl.ds(crow, NCH)], ti_v)
    plsc.subcore_barrier()
    ebase = wid * EPW
    _gather_stream(h_sh, ti_v, gz_hbm, rows_v, gsem, osem, ebase)


def _scatter_stream(val_hbm, idx_v, acc_sh, vbuf, lsem, ssem, ebase, nch):
    for b in range(NB):
        pltpu.async_copy(val_hbm.at[pl.ds(ebase + b * CH, CH)], vbuf.at[b],
                         lsem.at[b])

    def body(k, carry):
        @pl.when(k > 0)
        def _():
            pc = k - 1 + NB

            @pl.when(pc < nch)
            def _():
                pb = lax.rem(k - 1, NB)
                pltpu.make_async_copy(vbuf.at[pb],
                                      acc_sh.at[idx_v.at[k - 1]],
                                      ssem.at[pb]).wait()
                pltpu.async_copy(val_hbm.at[pl.ds(ebase + pc * CH, CH)],
                                 vbuf.at[pb], lsem.at[pb])

        b = lax.rem(k, NB)
        pltpu.make_async_copy(val_hbm.at[pl.ds(ebase + k * CH, CH)],
                              vbuf.at[b], lsem.at[b]).wait()
        pltpu.async_copy(vbuf.at[b], acc_sh.at[idx_v.at[k]], ssem.at[b],
                         add=True)
        return carry

    lax.fori_loop(0, nch, body, 0)

    def drain(b, carry):
        c = nch - NB + b
        cb = lax.rem(c, NB)
        pltpu.make_async_copy(vbuf.at[cb], acc_sh.at[idx_v.at[c]],
                              ssem.at[cb]).wait()
        return carry

    lax.fori_loop(0, NB, drain, 0)


@functools.cache
def _scatter2_build():
    return pl.kernel(
        _scatter2_body,
        out_type=jax.ShapeDtypeStruct((NC, VP, 16), F32),
        mesh=_sc_mesh(),
        compiler_params=pltpu.CompilerParams(use_tc_tiling_on_sc=False),
        scratch_types=[
            pltpu.VMEM((NCH, CH), jnp.int32),
            pltpu.VMEM((NB, CH, 16), F32),
            pltpu.VMEM((ZR, 16), F32),
            pltpu.VMEM_SHARED((VP, 16), F32),
            pltpu.SemaphoreType.DMA((NB,)),
            pltpu.SemaphoreType.DMA((NB,)),
        ],
    )


def _scatter2_body(u_hbm, w_hbm, src_hbm, tgt_hbm, out_hbm,
                   idx_v, vbuf, stage_v, acc_sh, lsem, ssem):
    cid = lax.axis_index("c")
    sid = lax.axis_index("s")
    wid = sid * NC + cid

    def zb(i, carry):
        stage_v[i] = jnp.zeros((16,), F32)
        return carry

    lax.fori_loop(0, ZR, zb, 0)
    pltpu.sync_copy(stage_v, acc_sh.at[pl.ds(sid * ZR, ZR)])
    plsc.subcore_barrier()

    ebase = wid * EPW
    nch = jnp.minimum((E_ - ebase) // CH, NCH)
    crow = wid * NCH

    pltpu.sync_copy(src_hbm.at[pl.ds(crow, NCH)], idx_v)
    _scatter_stream(u_hbm, idx_v, acc_sh, vbuf, lsem, ssem, ebase, nch)
    pltpu.sync_copy(tgt_hbm.at[pl.ds(crow, NCH)], idx_v)
    _scatter_stream(w_hbm, idx_v, acc_sh, vbuf, lsem, ssem, ebase, nch)

    plsc.subcore_barrier()
    pltpu.sync_copy(acc_sh.at[pl.ds(sid * ZR, ZR)], stage_v)
    pltpu.sync_copy(stage_v, out_hbm.at[cid, pl.ds(sid * ZR, ZR)])


def _h0_call(xpk, wbd, b8):
    return pl.pallas_call(
        _h0_body,
        out_shape=jax.ShapeDtypeStruct((VP // 8, 128), F32),
    )(xpk, wbd, b8)


def _maps_mv1_call(gs8, gt8, w1a, w1b, w2, pz, ru, pu, rw):
    zero2 = lambda i: (0, 0)
    return pl.pallas_call(
        _maps_mv1_body,
        grid=(GRID_E,),
        in_specs=[
            pl.BlockSpec((BLK_E // 8, 128), lambda i: (i, 0)),
            pl.BlockSpec((BLK_E // 8, 128), lambda i: (i, 0)),
            pl.BlockSpec((128, 512), zero2),
            pl.BlockSpec((128, 512), zero2),
            pl.BlockSpec((512, 512), zero2),
            pl.BlockSpec((128, 512), zero2),
            pl.BlockSpec((512, 128), zero2),
            pl.BlockSpec((128, 512), zero2),
            pl.BlockSpec((512, 128), zero2),
        ],
        out_specs=[
            pl.BlockSpec((BLK_E // 8, 512), lambda i: (i, 0)),
            pl.BlockSpec((BLK_E // 8, 128), lambda i: (i, 0)),
            pl.BlockSpec((BLK_E // 8, 128), lambda i: (i, 0)),
        ],
        out_shape=[
            jax.ShapeDtypeStruct((E_ // 8, 512), F32),
            jax.ShapeDtypeStruct((E_ // 8, 128), F32),
            jax.ShapeDtypeStruct((E_ // 8, 128), F32),
        ],
    )(gs8, gt8, w1a, w1b, w2, pz, ru, pu, rw)


def _mv2_call(mt8, gz8, pz, ru, pu, rw):
    zero2 = lambda i: (0, 0)
    return pl.pallas_call(
        _mv2_body,
        grid=(GRID_E,),
        in_specs=[
            pl.BlockSpec((BLK_E // 8, 512), lambda i: (i, 0)),
            pl.BlockSpec((BLK_E // 8, 128), lambda i: (i, 0)),
            pl.BlockSpec((128, 512), zero2),
            pl.BlockSpec((512, 128), zero2),
            pl.BlockSpec((128, 512), zero2),
            pl.BlockSpec((512, 128), zero2),
        ],
        out_specs=[
            pl.BlockSpec((BLK_E // 8, 128), lambda i: (i, 0)),
            pl.BlockSpec((BLK_E // 8, 128), lambda i: (i, 0)),
        ],
        out_shape=[
            jax.ShapeDtypeStruct((E_ // 8, 128), F32),
            jax.ShapeDtypeStruct((E_ // 8, 128), F32),
        ],
    )(mt8, gz8, pz, ru, pu, rw)


def _cmb_call(acc8, h8, wl, bl):
    return pl.pallas_call(
        _cmb_body,
        grid=(GRID_V,),
        in_specs=[
            pl.BlockSpec((NC, BLK_V // 8, 128), lambda i: (0, i, 0)),
            pl.BlockSpec((BLK_V // 8, 128), lambda i: (i, 0)),
            pl.BlockSpec((128, 128), lambda i: (0, 0)),
            pl.BlockSpec((1, 128), lambda i: (0, 0)),
        ],
        out_specs=pl.BlockSpec((BLK_V // 8, 128), lambda i: (i, 0)),
        out_shape=jax.ShapeDtypeStruct((VP // 8, 128), F32),
    )(acc8, h8, wl, bl)


def _cmb_out_call(acc8, h8, wl, bl, wo, bo):
    return pl.pallas_call(
        _cmb_out_body,
        grid=(GRID_V,),
        in_specs=[
            pl.BlockSpec((NC, BLK_V // 8, 128), lambda i: (0, i, 0)),
            pl.BlockSpec((BLK_V // 8, 128), lambda i: (i, 0)),
            pl.BlockSpec((128, 128), lambda i: (0, 0)),
            pl.BlockSpec((1, 128), lambda i: (0, 0)),
            pl.BlockSpec((128, 512), lambda i: (0, 0)),
            pl.BlockSpec((1, 512), lambda i: (0, 0)),
        ],
        out_specs=pl.BlockSpec((BLK_V // 8, 512), lambda i: (i, 0)),
        out_shape=jax.ShapeDtypeStruct((VP // 8, 512), F32),
    )(acc8, h8, wl, bl, wo, bo)


_PERM = np.array([(c % 8) * 8 + c // 8 for c in range(64)])

_PZ = np.zeros((16, 64), np.float32)
_RU = np.zeros((64, 16), np.float32)
_PU = np.zeros((16, 64), np.float32)
_RW = np.zeros((64, 16), np.float32)
for _j in range(8):
    for _i in range(8):
        _PZ[_j, 8 * _j + _i] = 1.0
        _RU[8 * _j + _i, _i] = 1.0
        _PU[_i, 8 * _j + _i] = 1.0
        _RW[8 * _j + _i, _j] = 1.0
_I8 = np.eye(8, dtype=np.float32)
_PZ8 = np.kron(_I8, _PZ)
_RU8 = np.kron(_I8, _RU)
_PU8 = np.kron(_I8, _PU)
_RW8 = np.kron(_I8, _RW)


def kernel(x, edge_index, W_in, b_in, W1, W2, Ws1, bs1, Ws2, bs2, Wout, bout):
    src = edge_index[0].astype(jnp.int32)
    tgt = edge_index[1].astype(jnp.int32)
    src2 = jnp.pad(src, (0, EPAD - E_)).reshape(NW * NCH, CH)
    tgt2 = jnp.pad(tgt, (0, EPAD - E_)).reshape(NW * NCH, CH)

    i8 = jnp.asarray(_I8)
    W_in16 = jnp.pad(W_in, ((0, 0), (0, 8)))
    W_in_bd = jnp.kron(i8, W_in16)
    b_in8 = jnp.tile(jnp.pad(b_in, (0, 8)).reshape(1, 16), (1, 8))
    W1a16 = jnp.pad(W1[:8], ((0, 8), (0, 0)))
    W1b16 = jnp.pad(W1[8:], ((0, 8), (0, 0)))
    W1A = jnp.kron(i8, W1a16)
    W1B = jnp.kron(i8, W1b16)
    W2BD = jnp.kron(i8, W2[:, _PERM])
    WL1 = jnp.kron(i8, jnp.pad(Ws1, ((0, 8), (0, 8))))
    WL2 = jnp.kron(i8, jnp.pad(Ws2, ((0, 8), (0, 8))))
    bl1_8 = jnp.tile(jnp.pad(bs1, (0, 8)).reshape(1, 16), (1, 8))
    bl2_8 = jnp.tile(jnp.pad(bs2, (0, 8)).reshape(1, 16), (1, 8))
    WOBD = jnp.kron(i8, jnp.pad(Wout, ((0, 8), (0, 0))))
    bo8 = jnp.tile(bout.reshape(1, 64), (1, 8))
    xpk = jnp.pad(x, ((0, VP - V_), (0, 0))).reshape(VP // 8, 1024)

    h8 = _h0_call(xpk, W_in_bd, b_in8)
    h16 = jnp.reshape(h8, (VP, 16))
    gs, gt = _gather2_build()(h16, src2, tgt2)
    gs8 = jnp.reshape(gs, (EPAD // 8, 128))
    gt8 = jnp.reshape(gt, (EPAD // 8, 128))
    pz8, ru8, pu8, rw8 = (jnp.asarray(_PZ8), jnp.asarray(_RU8),
                          jnp.asarray(_PU8), jnp.asarray(_RW8))
    mt8, U1, W1v = _maps_mv1_call(gs8, gt8, W1A, W1B, W2BD,
                                  pz8, ru8, pu8, rw8)
    acc1 = _scatter2_build()(jnp.reshape(U1, (E_, 16)),
                             jnp.reshape(W1v, (E_, 16)), src2, tgt2)
    acc1_8 = jnp.reshape(acc1, (NC, VP // 8, 128))
    h1_8 = _cmb_call(acc1_8, h8, WL1, bl1_8)
    h1 = jnp.reshape(h1_8, (VP, 16))
    gz2 = _gather1_build()(h1, tgt2)
    gz2_8 = jnp.reshape(gz2, (EPAD // 8, 128))
    U2, W2v = _mv2_call(mt8, gz2_8, pz8, ru8, pu8, rw8)
    acc2 = _scatter2_build()(jnp.reshape(U2, (E_, 16)),
                             jnp.reshape(W2v, (E_, 16)), src2, tgt2)
    acc2_8 = jnp.reshape(acc2, (NC, VP // 8, 128))
    outp = _cmb_out_call(acc2_8, h1_8, WL2, bl2_8, WOBD, bo8)
    return jnp.reshape(outp, (VP, 64))[:V_]


# speedup vs baseline: 120.7616x; 1.2925x over previous
import functools

import numpy as np
import jax
import jax.numpy as jnp
from jax import lax
from jax.experimental import pallas as pl
from jax.experimental.pallas import tpu as pltpu
from jax.experimental.pallas import tpu_sc as plsc

F32 = jnp.float32

NC, NS = 2, 16
NW = NC * NS
V_, E_ = 10000, 320000
VP = 10240
EPAD = 327680
CH = 128
EPW = EPAD // NW
NCH = EPW // CH
ZR = VP // NS
BLK_E = 8000
GRID_E = E_ // BLK_E
BLK_V = 2048
GRID_V = VP // BLK_V


@functools.cache
def _sc_mesh():
    return plsc.VectorSubcoreMesh(core_axis_name="c", subcore_axis_name="s",
                                  num_cores=NC, num_subcores=NS)


def _h0_body(x_ref, w_ref, b_ref, o_ref):
    o_ref[...] = (
        jnp.dot(x_ref[...], w_ref[...], preferred_element_type=F32) + b_ref[...]
    )


def _edge_matvec8(m8, gz8, pz_ref, ru_ref, pu_ref, rw_ref):
    zrep = jnp.dot(gz8, pz_ref[...], preferred_element_type=F32)
    u8 = jnp.dot(m8 * zrep, ru_ref[...], preferred_element_type=F32)
    urep = jnp.dot(u8, pu_ref[...], preferred_element_type=F32)
    w8 = jnp.dot(m8 * urep, rw_ref[...], preferred_element_type=F32)
    return -u8, w8


def _maps_mv1_body(gs_ref, gt_ref, w1a_ref, w1b_ref, w2_ref,
                   pz_ref, ru_ref, pu_ref, rw_ref, mt_ref, u_ref, w_ref):
    gs8 = gs_ref[...]
    gt8 = gt_ref[...]
    hid8 = jnp.tanh(
        jnp.dot(gs8, w1a_ref[...], preferred_element_type=F32)
        + jnp.dot(gt8, w1b_ref[...], preferred_element_type=F32))
    m8 = jnp.tanh(jnp.dot(hid8, w2_ref[...], preferred_element_type=F32))
    mt_ref[...] = m8
    U8, W8 = _edge_matvec8(m8, gt8, pz_ref, ru_ref, pu_ref, rw_ref)
    u_ref[...] = U8
    w_ref[...] = W8


def _mv2_body(mt_ref, gz_ref, pz_ref, ru_ref, pu_ref, rw_ref, u_ref, w_ref):
    U8, W8 = _edge_matvec8(mt_ref[...], gz_ref[...],
                           pz_ref, ru_ref, pu_ref, rw_ref)
    u_ref[...] = U8
    w_ref[...] = W8


def _cmb_body(acc_ref, h_ref, wl_ref, bl_ref, o_ref):
    a = acc_ref[0] + acc_ref[1]
    o_ref[...] = jnp.maximum(
        a + jnp.dot(h_ref[...], wl_ref[...], preferred_element_type=F32)
        + bl_ref[...], 0.0)


def _cmb_out_body(acc_ref, h_ref, wl_ref, bl_ref, wo_ref, bo_ref, o_ref):
    a = acc_ref[0] + acc_ref[1]
    h2 = jnp.maximum(
        a + jnp.dot(h_ref[...], wl_ref[...], preferred_element_type=F32)
        + bl_ref[...], 0.0)
    o_ref[...] = jnp.dot(h2, wo_ref[...], preferred_element_type=F32) + bo_ref[...]


NB = 8


def _gather_stream(h_hbm, idx_v, out_hbm, rows_v, gsem, osem, ebase):
    for b in range(NB):
        pltpu.async_copy(h_hbm.at[idx_v.at[b]], rows_v.at[b], gsem.at[b])

    def body(k, carry):
        @pl.when(k > 0)
        def _():
            pc = k - 1 + NB

            @pl.when(pc < NCH)
            def _():
                pb = lax.rem(k - 1, NB)
                pltpu.make_async_copy(
                    rows_v.at[pb],
                    out_hbm.at[pl.ds(ebase + (k - 1) * CH, CH)],
                    osem.at[pb]).wait()
                pltpu.async_copy(h_hbm.at[idx_v.at[pc]], rows_v.at[pb],
                                 gsem.at[pb])

        b = lax.rem(k, NB)
        pltpu.make_async_copy(h_hbm.at[idx_v.at[k]], rows_v.at[b],
                              gsem.at[b]).wait()
        pltpu.async_copy(rows_v.at[b], out_hbm.at[pl.ds(ebase + k * CH, CH)],
                         osem.at[b])
        return carry

    lax.fori_loop(0, NCH, body, 0)

    def drain(b, carry):
        c = NCH - NB + b
        pltpu.make_async_copy(rows_v.at[b],
                              out_hbm.at[pl.ds(ebase + c * CH, CH)],
                              osem.at[b]).wait()
        return carry

    lax.fori_loop(0, NB, drain, 0)


@functools.cache
def _gather2_build():
    return pl.kernel(
        _gather2_body,
        out_type=(jax.ShapeDtypeStruct((EPAD, 16), F32),
                  jax.ShapeDtypeStruct((EPAD, 16), F32)),
        mesh=_sc_mesh(),
        compiler_params=pltpu.CompilerParams(use_tc_tiling_on_sc=False),
        scratch_types=[
            pltpu.VMEM((NCH, CH), jnp.int32),
            pltpu.VMEM((NCH, CH), jnp.int32),
            pltpu.VMEM((NB, CH, 16), F32),
            pltpu.VMEM_SHARED((VP, 16), F32),
            pltpu.SemaphoreType.DMA((NB,)),
            pltpu.SemaphoreType.DMA((NB,)),
        ],
    )


def _gather2_body(h_hbm, src_hbm, tgt_hbm, gs_hbm, gt_hbm,
                  si_v, ti_v, rows_v, h_sh, gsem, osem):
    sid = lax.axis_index("s")
    wid = sid * NC + lax.axis_index("c")
    crow = wid * NCH
    pltpu.sync_copy(h_hbm.at[pl.ds(sid * ZR, ZR)], h_sh.at[pl.ds(sid * ZR, ZR)])
    pltpu.sync_copy(src_hbm.at[pl.ds(crow, NCH)], si_v)
    pltpu.sync_copy(tgt_hbm.at[pl.ds(crow, NCH)], ti_v)
    plsc.subcore_barrier()
    ebase = wid * EPW
    _gather_stream(h_sh, si_v, gs_hbm, rows_v, gsem, osem, ebase)
    _gather_stream(h_sh, ti_v, gt_hbm, rows_v, gsem, osem, ebase)


@functools.cache
def _gather1_build():
    return pl.kernel(
        _gather1_body,
        out_type=jax.ShapeDtypeStruct((EPAD, 16), F32),
        mesh=_sc_mesh(),
        compiler_params=pltpu.CompilerParams(use_tc_tiling_on_sc=False),
        scratch_types=[
            pltpu.VMEM((NCH, CH), jnp.int32),
            pltpu.VMEM((NB, CH, 16), F32),
            pltpu.VMEM_SHARED((VP, 16), F32),
            pltpu.SemaphoreType.DMA((NB,)),
            pltpu.SemaphoreType.DMA((NB,)),
        ],
    )


def _gather1_body(h_hbm, tgt_hbm, gz_hbm, ti_v, rows_v, h_sh, gsem, osem):
    sid = lax.axis_index("s")
    wid = sid * NC + lax.axis_index("c")
    crow = wid * NCH
    pltpu.sync_copy(h_hbm.at[pl.ds(sid * ZR, ZR)], h_sh.at[pl.ds(sid * ZR, ZR)])
    pltpu.sync_copy(tgt_hbm.at[pl.ds(crow, NCH)], ti_v)
    plsc.subcore_barrier()
    ebase = wid * EPW
    _gather_stream(h_sh, ti_v, gz_hbm, rows_v, gsem, osem, ebase)


def _scatter_stream(val_hbm, idx_v, acc_sh, vbuf, lsem, ssem, ebase, nch):
    for b in range(NB):
        pltpu.async_copy(val_hbm.at[pl.ds(ebase + b * CH, CH)], vbuf.at[b],
                         lsem.at[b])

    def body(k, carry):
        @pl.when(k > 0)
        def _():
            pc = k - 1 + NB

            @pl.when(pc < nch)
            def _():
                pb = lax.rem(k - 1, NB)
                pltpu.make_async_copy(vbuf.at[pb],
                                      acc_sh.at[idx_v.at[k - 1]],
                                      ssem.at[pb]).wait()
                pltpu.async_copy(val_hbm.at[pl.ds(ebase + pc * CH, CH)],
                                 vbuf.at[pb], lsem.at[pb])

        b = lax.rem(k, NB)
        pltpu.make_async_copy(val_hbm.at[pl.ds(ebase + k * CH, CH)],
                              vbuf.at[b], lsem.at[b]).wait()
        pltpu.async_copy(vbuf.at[b], acc_sh.at[idx_v.at[k]], ssem.at[b],
                         add=True)
        return carry

    lax.fori_loop(0, nch, body, 0)

    def drain(b, carry):
        c = nch - NB + b
        cb = lax.rem(c, NB)
        pltpu.make_async_copy(vbuf.at[cb], acc_sh.at[idx_v.at[c]],
                              ssem.at[cb]).wait()
        return carry

    lax.fori_loop(0, NB, drain, 0)


@functools.cache
def _scatter2_build():
    return pl.kernel(
        _scatter2_body,
        out_type=jax.ShapeDtypeStruct((NC, VP, 16), F32),
        mesh=_sc_mesh(),
        compiler_params=pltpu.CompilerParams(use_tc_tiling_on_sc=False),
        scratch_types=[
            pltpu.VMEM((NCH, CH), jnp.int32),
            pltpu.VMEM((NB, CH, 16), F32),
            pltpu.VMEM((ZR, 16), F32),
            pltpu.VMEM_SHARED((VP, 16), F32),
            pltpu.SemaphoreType.DMA((NB,)),
            pltpu.SemaphoreType.DMA((NB,)),
        ],
    )


def _scatter2_body(u_hbm, w_hbm, src_hbm, tgt_hbm, out_hbm,
                   idx_v, vbuf, stage_v, acc_sh, lsem, ssem):
    cid = lax.axis_index("c")
    sid = lax.axis_index("s")
    wid = sid * NC + cid

    def zb(i, carry):
        stage_v[i] = jnp.zeros((16,), F32)
        return carry

    lax.fori_loop(0, ZR, zb, 0)
    pltpu.sync_copy(stage_v, acc_sh.at[pl.ds(sid * ZR, ZR)])
    plsc.subcore_barrier()

    ebase = wid * EPW
    nch = jnp.minimum((E_ - ebase) // CH, NCH)
    crow = wid * NCH

    pltpu.sync_copy(src_hbm.at[pl.ds(crow, NCH)], idx_v)
    _scatter_stream(u_hbm, idx_v, acc_sh, vbuf, lsem, ssem, ebase, nch)
    pltpu.sync_copy(tgt_hbm.at[pl.ds(crow, NCH)], idx_v)
    _scatter_stream(w_hbm, idx_v, acc_sh, vbuf, lsem, ssem, ebase, nch)

    plsc.subcore_barrier()
    pltpu.sync_copy(acc_sh.at[pl.ds(sid * ZR, ZR)], stage_v)
    pltpu.sync_copy(stage_v, out_hbm.at[cid, pl.ds(sid * ZR, ZR)])


def _h0_call(xpk, wbd, b8):
    return pl.pallas_call(
        _h0_body,
        out_shape=jax.ShapeDtypeStruct((VP // 8, 128), F32),
    )(xpk, wbd, b8)


def _maps_mv1_call(gs8, gt8, w1a, w1b, w2, pz, ru, pu, rw):
    zero2 = lambda i: (0, 0)
    return pl.pallas_call(
        _maps_mv1_body,
        grid=(GRID_E,),
        in_specs=[
            pl.BlockSpec((BLK_E // 8, 128), lambda i: (i, 0)),
            pl.BlockSpec((BLK_E // 8, 128), lambda i: (i, 0)),
            pl.BlockSpec((128, 512), zero2),
            pl.BlockSpec((128, 512), zero2),
            pl.BlockSpec((512, 512), zero2),
            pl.BlockSpec((128, 512), zero2),
            pl.BlockSpec((512, 128), zero2),
            pl.BlockSpec((128, 512), zero2),
            pl.BlockSpec((512, 128), zero2),
        ],
        out_specs=[
            pl.BlockSpec((BLK_E // 8, 512), lambda i: (i, 0)),
            pl.BlockSpec((BLK_E // 8, 128), lambda i: (i, 0)),
            pl.BlockSpec((BLK_E // 8, 128), lambda i: (i, 0)),
        ],
        out_shape=[
            jax.ShapeDtypeStruct((E_ // 8, 512), F32),
            jax.ShapeDtypeStruct((E_ // 8, 128), F32),
            jax.ShapeDtypeStruct((E_ // 8, 128), F32),
        ],
    )(gs8, gt8, w1a, w1b, w2, pz, ru, pu, rw)


def _mv2_call(mt8, gz8, pz, ru, pu, rw):
    zero2 = lambda i: (0, 0)
    return pl.pallas_call(
        _mv2_body,
        grid=(GRID_E,),
        in_specs=[
            pl.BlockSpec((BLK_E // 8, 512), lambda i: (i, 0)),
            pl.BlockSpec((BLK_E // 8, 128), lambda i: (i, 0)),
            pl.BlockSpec((128, 512), zero2),
            pl.BlockSpec((512, 128), zero2),
            pl.BlockSpec((128, 512), zero2),
            pl.BlockSpec((512, 128), zero2),
        ],
        out_specs=[
            pl.BlockSpec((BLK_E // 8, 128), lambda i: (i, 0)),
            pl.BlockSpec((BLK_E // 8, 128), lambda i: (i, 0)),
        ],
        out_shape=[
            jax.ShapeDtypeStruct((E_ // 8, 128), F32),
            jax.ShapeDtypeStruct((E_ // 8, 128), F32),
        ],
    )(mt8, gz8, pz, ru, pu, rw)


def _cmb_call(acc8, h8, wl, bl):
    return pl.pallas_call(
        _cmb_body,
        grid=(GRID_V,),
        in_specs=[
            pl.BlockSpec((NC, BLK_V // 8, 128), lambda i: (0, i, 0)),
            pl.BlockSpec((BLK_V // 8, 128), lambda i: (i, 0)),
            pl.BlockSpec((128, 128), lambda i: (0, 0)),
            pl.BlockSpec((1, 128), lambda i: (0, 0)),
        ],
        out_specs=pl.BlockSpec((BLK_V // 8, 128), lambda i: (i, 0)),
        out_shape=jax.ShapeDtypeStruct((VP // 8, 128), F32),
    )(acc8, h8, wl, bl)


def _cmb_out_call(acc8, h8, wl, bl, wo, bo):
    return pl.pallas_call(
        _cmb_out_body,
        grid=(GRID_V,),
        in_specs=[
            pl.BlockSpec((NC, BLK_V // 8, 128), lambda i: (0, i, 0)),
            pl.BlockSpec((BLK_V // 8, 128), lambda i: (i, 0)),
            pl.BlockSpec((128, 128), lambda i: (0, 0)),
            pl.BlockSpec((1, 128), lambda i: (0, 0)),
            pl.BlockSpec((128, 512), lambda i: (0, 0)),
            pl.BlockSpec((1, 512), lambda i: (0, 0)),
        ],
        out_specs=pl.BlockSpec((BLK_V // 8, 512), lambda i: (i, 0)),
        out_shape=jax.ShapeDtypeStruct((VP // 8, 512), F32),
    )(acc8, h8, wl, bl, wo, bo)


_PERM = np.array([(c % 8) * 8 + c // 8 for c in range(64)])

_PZ = np.zeros((16, 64), np.float32)
_RU = np.zeros((64, 16), np.float32)
_PU = np.zeros((16, 64), np.float32)
_RW = np.zeros((64, 16), np.float32)
for _j in range(8):
    for _i in range(8):
        _PZ[_j, 8 * _j + _i] = 1.0
        _RU[8 * _j + _i, _i] = 1.0
        _PU[_i, 8 * _j + _i] = 1.0
        _RW[8 * _j + _i, _j] = 1.0
_I8 = np.eye(8, dtype=np.float32)
_PZ8 = np.kron(_I8, _PZ)
_RU8 = np.kron(_I8, _RU)
_PU8 = np.kron(_I8, _PU)
_RW8 = np.kron(_I8, _RW)


def kernel(x, edge_index, W_in, b_in, W1, W2, Ws1, bs1, Ws2, bs2, Wout, bout):
    src = edge_index[0].astype(jnp.int32)
    tgt = edge_index[1].astype(jnp.int32)
    src2 = jnp.pad(src, (0, EPAD - E_)).reshape(NW * NCH, CH)
    tgt2 = jnp.pad(tgt, (0, EPAD - E_)).reshape(NW * NCH, CH)

    i8 = jnp.asarray(_I8)
    W_in16 = jnp.pad(W_in, ((0, 0), (0, 8)))
    W_in_bd = jnp.kron(i8, W_in16)
    b_in8 = jnp.tile(jnp.pad(b_in, (0, 8)).reshape(1, 16), (1, 8))
    W1a16 = jnp.pad(W1[:8], ((0, 8), (0, 0)))
    W1b16 = jnp.pad(W1[8:], ((0, 8), (0, 0)))
    W1A = jnp.kron(i8, W1a16)
    W1B = jnp.kron(i8, W1b16)
    W2BD = jnp.kron(i8, W2[:, _PERM])
    WL1 = jnp.kron(i8, jnp.pad(Ws1, ((0, 8), (0, 8))))
    WL2 = jnp.kron(i8, jnp.pad(Ws2, ((0, 8), (0, 8))))
    bl1_8 = jnp.tile(jnp.pad(bs1, (0, 8)).reshape(1, 16), (1, 8))
    bl2_8 = jnp.tile(jnp.pad(bs2, (0, 8)).reshape(1, 16), (1, 8))
    WOBD = jnp.kron(i8, jnp.pad(Wout, ((0, 8), (0, 0))))
    bo8 = jnp.tile(bout.reshape(1, 64), (1, 8))
    xpk = jnp.pad(x, ((0, VP - V_), (0, 0))).reshape(VP // 8, 1024)

    h8 = _h0_call(xpk, W_in_bd, b_in8)
    h16 = jnp.reshape(h8, (VP, 16))
    gs, gt = _gather2_build()(h16, src2, tgt2)
    gs8 = jnp.reshape(gs, (EPAD // 8, 128))
    gt8 = jnp.reshape(gt, (EPAD // 8, 128))
    pz8, ru8, pu8, rw8 = (jnp.asarray(_PZ8), jnp.asarray(_RU8),
                          jnp.asarray(_PU8), jnp.asarray(_RW8))
    mt8, U1, W1v = _maps_mv1_call(gs8, gt8, W1A, W1B, W2BD,
                                  pz8, ru8, pu8, rw8)
    acc1 = _scatter2_build()(jnp.reshape(U1, (E_, 16)),
                             jnp.reshape(W1v, (E_, 16)), src2, tgt2)
    acc1_8 = jnp.reshape(acc1, (NC, VP // 8, 128))
    h1_8 = _cmb_call(acc1_8, h8, WL1, bl1_8)
    h1 = jnp.reshape(h1_8, (VP, 16))
    gz2 = _gather1_build()(h1, tgt2)
    gz2_8 = jnp.reshape(gz2, (EPAD // 8, 128))
    U2, W2v = _mv2_call(mt8, gz2_8, pz8, ru8, pu8, rw8)
    acc2 = _scatter2_build()(jnp.reshape(U2, (E_, 16)),
                             jnp.reshape(W2v, (E_, 16)), src2, tgt2)
    acc2_8 = jnp.reshape(acc2, (NC, VP // 8, 128))
    outp = _cmb_out_call(acc2_8, h1_8, WL2, bl2_8, WOBD, bo8)
    return jnp.reshape(outp, (VP, 64))[:V_]
